# Initial kernel scaffold; baseline (speedup 1.0000x reference)
import jax
import jax.numpy as jnp
from jax.experimental import pallas as pl


def kernel(node_features, edge_index, edge_features, W_e1, b_e1, g_e1, be_e1, W_e2, b_e2, g_e2, be_e2, W_n1, b_n1, g_n1, be_n1, W_n2, b_n2, g_n2, be_n2):
    raise NotImplementedError("write your pallas kernel here")


# speedup vs baseline: 1.9151x; 1.9151x over previous
import functools

import jax
import jax.numpy as jnp
from jax import lax
from jax.experimental import pallas as pl
from jax.experimental.pallas import tpu as pltpu
from jax.experimental.pallas import tpu_sc as plsc

F32 = jnp.float32
_PREC = lax.Precision.HIGHEST

NC = 2
NS = 16
NW = NC * NS


def _ln(h, g, b, eps=1e-5):
    m = jnp.mean(h, axis=-1, keepdims=True)
    v = jnp.mean((h - m) ** 2, axis=-1, keepdims=True)
    return (h - m) / jnp.sqrt(v + eps) * g + b


def _gelu(h):
    return 0.5 * h * (1.0 + lax.erf(h * (2.0 ** -0.5)))


def _proj_body(x_ref, wa_ref, wb_ref, a_ref, b_ref):
    x = x_ref[...]
    a_ref[...] = jnp.dot(x, wa_ref[...], preferred_element_type=F32,
                         precision=_PREC)
    b_ref[...] = jnp.dot(x, wb_ref[...], preferred_element_type=F32,
                         precision=_PREC)


def _node_proj(x, wa, wb, bn):
    n, d = x.shape
    h = wa.shape[1]
    return pl.pallas_call(
        _proj_body,
        grid=(n // bn,),
        in_specs=[pl.BlockSpec((bn, d), lambda i: (i, 0)),
                  pl.BlockSpec((d, h), lambda i: (0, 0)),
                  pl.BlockSpec((d, h), lambda i: (0, 0))],
        out_specs=[pl.BlockSpec((bn, h), lambda i: (i, 0)),
                   pl.BlockSpec((bn, h), lambda i: (i, 0))],
        out_shape=[jax.ShapeDtypeStruct((n, h), F32),
                   jax.ShapeDtypeStruct((n, h), F32)],
    )(x, wa, wb)


def _sc_gather(x, aproj, bproj, row, col, cb=80):
    n, d = x.shape
    e = row.shape[0]
    eh = aproj.shape[1]
    per_w = e // NW
    n_chunks = per_w // cb
    mesh = plsc.VectorSubcoreMesh(core_axis_name="c", subcore_axis_name="s")

    @functools.partial(
        pl.kernel, mesh=mesh,
        compiler_params=pltpu.CompilerParams(use_tc_tiling_on_sc=False),
        out_type=[jax.ShapeDtypeStruct((e, eh), F32),
                  jax.ShapeDtypeStruct((e, 16), F32)],
        scratch_types=[
            pltpu.VMEM((cb,), jnp.int32),
            pltpu.VMEM((cb,), jnp.int32),
            pltpu.VMEM((cb, d), F32),
            pltpu.VMEM((cb, d), F32),
            pltpu.VMEM((cb, eh), F32),
            pltpu.VMEM((cb, eh), F32),
            pltpu.VMEM((cb, eh), F32),
            pltpu.VMEM((cb, 16), F32),
            pltpu.SemaphoreType.DMA,
            pltpu.SemaphoreType.DMA,
            pltpu.SemaphoreType.DMA,
            pltpu.SemaphoreType.DMA,
        ],
    )
    def k(x_hbm, ap_hbm, bp_hbm, row_hbm, col_hbm, hp_hbm, dp_hbm,
          ir_v, ic_v, xr_v, xc_v, pa_v, pb_v, oh_v, od_v, s0, s1, s2, s3):
        wid = lax.axis_index("s") * NC + lax.axis_index("c")

        def chunk(ci, _):
            base = wid * per_w + ci * cb
            pltpu.sync_copy(row_hbm.at[pl.ds(base, cb)], ir_v)
            pltpu.sync_copy(col_hbm.at[pl.ds(base, cb)], ic_v)
            cp0 = pltpu.async_copy(x_hbm.at[ir_v], xr_v, s0)
            cp1 = pltpu.async_copy(x_hbm.at[ic_v], xc_v, s1)
            cp2 = pltpu.async_copy(ap_hbm.at[ir_v], pa_v, s2)
            cp3 = pltpu.async_copy(bp_hbm.at[ic_v], pb_v, s3)
            cp0.wait()
            cp1.wait()
            cp2.wait()
            cp3.wait()

            def edge(ei, _):
                acc = jnp.zeros((16,), F32)
                for j in range(d // 16):
                    da = (xr_v[ei, pl.ds(j * 16, 16)]
                          - xc_v[ei, pl.ds(j * 16, 16)])
                    acc = acc + da * da
                od_v[ei, :] = acc
                for j in range(eh // 16):
                    oh_v[ei, pl.ds(j * 16, 16)] = (
                        pa_v[ei, pl.ds(j * 16, 16)]
                        + pb_v[ei, pl.ds(j * 16, 16)])
                return 0

            lax.fori_loop(0, cb, edge, 0)
            pltpu.sync_copy(oh_v, hp_hbm.at[pl.ds(base, cb)])
            pltpu.sync_copy(od_v, dp_hbm.at[pl.ds(base, cb)])
            return 0

        lax.fori_loop(0, n_chunks, chunk, 0)

    return k(x, aproj, bproj, row, col)


def _edge_body(hp_ref, dp_ref, ef_ref, wef_ref, aux1_ref, we2_ref, aux2_ref,
               out_ref):
    d2 = jnp.sum(dp_ref[...], axis=1, keepdims=True)
    dist = jnp.sqrt(d2 + 1e-12)
    aux1 = aux1_ref[...]
    h = (hp_ref[...]
         + jnp.dot(ef_ref[...], wef_ref[...], preferred_element_type=F32,
                   precision=_PREC)
         + d2 * aux1[0:1] + dist * aux1[1:2] + aux1[2:3])
    h = _gelu(_ln(h, aux1[3:4], aux1[4:5]))
    aux2 = aux2_ref[...]
    h2 = jnp.dot(h, we2_ref[...], preferred_element_type=F32,
                 precision=_PREC) + aux2[0:1]
    out_ref[...] = _gelu(_ln(h2, aux2[1:2], aux2[2:3]))


def _edge_mlp(hpart, d2part, ef, wef, aux1, we2, aux2, be=4000):
    e, eh = hpart.shape
    de = ef.shape[1]
    oe = we2.shape[1]
    return pl.pallas_call(
        _edge_body,
        grid=(e // be,),
        in_specs=[pl.BlockSpec((be, eh), lambda i: (i, 0)),
                  pl.BlockSpec((be, 16), lambda i: (i, 0)),
                  pl.BlockSpec((be, de), lambda i: (i, 0)),
                  pl.BlockSpec((de, eh), lambda i: (0, 0)),
                  pl.BlockSpec((5, eh), lambda i: (0, 0)),
                  pl.BlockSpec((eh, oe), lambda i: (0, 0)),
                  pl.BlockSpec((3, oe), lambda i: (0, 0))],
        out_specs=pl.BlockSpec((be, oe), lambda i: (i, 0)),
        out_shape=jax.ShapeDtypeStruct((e, oe), F32),
    )(hpart, d2part, ef, wef, aux1, we2, aux2)


def _sc_scatter(e_arr, col, zeros_init, cb=80):
    e, oe = e_arr.shape
    n = zeros_init.shape[0]
    per_w = e // NW
    n_chunks = per_w // cb
    rows_per_tile = n // NS
    mesh = plsc.VectorSubcoreMesh(core_axis_name="c", subcore_axis_name="s")

    @functools.partial(
        pl.kernel, mesh=mesh,
        compiler_params=pltpu.CompilerParams(use_tc_tiling_on_sc=False),
        out_type=jax.ShapeDtypeStruct((NC * n, oe), F32),
        scratch_types=[
            pltpu.VMEM_SHARED((n, oe), F32),
            pltpu.VMEM((cb,), jnp.int32),
            pltpu.VMEM((cb, oe), F32),
        ],
    )
    def k(e_hbm, col_hbm, z_hbm, out_hbm, shared, ic_v, eb_v):
        cid = lax.axis_index("c")
        sid = lax.axis_index("s")
        pltpu.sync_copy(z_hbm.at[pl.ds(sid * rows_per_tile, rows_per_tile)],
                        shared.at[pl.ds(sid * rows_per_tile, rows_per_tile)])
        plsc.subcore_barrier()
        wid = sid * NC + cid

        def chunk(ci, _):
            base = wid * per_w + ci * cb
            pltpu.sync_copy(col_hbm.at[pl.ds(base, cb)], ic_v)
            pltpu.sync_copy(e_hbm.at[pl.ds(base, cb)], eb_v)
            pltpu.sync_copy(eb_v, shared.at[ic_v], add=True)
            return 0

        lax.fori_loop(0, n_chunks, chunk, 0)
        plsc.subcore_barrier()
        pltpu.sync_copy(
            shared.at[pl.ds(sid * rows_per_tile, rows_per_tile)],
            out_hbm.at[pl.ds(cid * n + sid * rows_per_tile, rows_per_tile)])

    return k(e_arr, col, zeros_init)


def _node_body(x_ref, p0_ref, p1_ref, wx_ref, wg_ref, aux1_ref, w2_ref,
               aux2_ref, out_ref):
    agg = p0_ref[...] + p1_ref[...]
    aux1 = aux1_ref[...]
    h = (jnp.dot(x_ref[...], wx_ref[...], preferred_element_type=F32,
                 precision=_PREC)
         + jnp.dot(agg, wg_ref[...], preferred_element_type=F32,
                   precision=_PREC)
         + aux1[0:1])
    h = _gelu(_ln(h, aux1[1:2], aux1[2:3]))
    aux2 = aux2_ref[...]
    h2 = jnp.dot(h, w2_ref[...], preferred_element_type=F32,
                 precision=_PREC) + aux2[0:1]
    out_ref[...] = _gelu(_ln(h2, aux2[1:2], aux2[2:3]))


def _node_mlp(x, p0, p1, wx, wg, aux1, w2, aux2, bn=2000):
    n, d = x.shape
    oe = p0.shape[1]
    nh = wx.shape[1]
    on = w2.shape[1]
    return pl.pallas_call(
        _node_body,
        grid=(n // bn,),
        in_specs=[pl.BlockSpec((bn, d), lambda i: (i, 0)),
                  pl.BlockSpec((bn, oe), lambda i: (i, 0)),
                  pl.BlockSpec((bn, oe), lambda i: (i, 0)),
                  pl.BlockSpec((d, nh), lambda i: (0, 0)),
                  pl.BlockSpec((oe, nh), lambda i: (0, 0)),
                  pl.BlockSpec((3, nh), lambda i: (0, 0)),
                  pl.BlockSpec((nh, on), lambda i: (0, 0)),
                  pl.BlockSpec((3, on), lambda i: (0, 0))],
        out_specs=pl.BlockSpec((bn, on), lambda i: (i, 0)),
        out_shape=jax.ShapeDtypeStruct((n, on), F32),
    )(x, p0, p1, wx, wg, aux1, w2, aux2)


def kernel(node_features, edge_index, edge_features,
           W_e1, b_e1, g_e1, be_e1, W_e2, b_e2, g_e2, be_e2,
           W_n1, b_n1, g_n1, be_n1, W_n2, b_n2, g_n2, be_n2):
    n, d = node_features.shape
    e = edge_index.shape[1]
    oe = W_e2.shape[1]

    ws, wt, wd = W_e1[:d], W_e1[d:2 * d], W_e1[2 * d:3 * d]
    w_d2, w_dist = W_e1[3 * d], W_e1[3 * d + 1]
    wef = W_e1[3 * d + 2:]
    wa = ws + wd
    wb = wt - wd
    row = edge_index[0]
    col = edge_index[1]

    aproj, bproj = _node_proj(node_features, wa, wb, bn=2000)
    hpart, d2part = _sc_gather(node_features, aproj, bproj, row, col)

    aux_e1 = jnp.stack([w_d2, w_dist, b_e1, g_e1, be_e1])
    aux_e2 = jnp.stack([b_e2, g_e2, be_e2])
    e_out = _edge_mlp(hpart, d2part, edge_features, wef, aux_e1, W_e2, aux_e2)

    parts = _sc_scatter(e_out, col, jnp.zeros((n, oe), F32))
    p0, p1 = parts[:n], parts[n:]

    aux_n1 = jnp.stack([b_n1, g_n1, be_n1])
    aux_n2 = jnp.stack([b_n2, g_n2, be_n2])
    n_out = _node_mlp(node_features, p0, p1, W_n1[:d], W_n1[d:],
                      aux_n1, W_n2, aux_n2)
    return (n_out, e_out)


# speedup vs baseline: 2.2585x; 1.1793x over previous
import functools

import jax
import jax.numpy as jnp
from jax import lax
from jax.experimental import pallas as pl
from jax.experimental.pallas import tpu as pltpu
from jax.experimental.pallas import tpu_sc as plsc

F32 = jnp.float32
_PREC = lax.Precision.HIGHEST

NC = 2
NS = 16
NW = NC * NS


def _ln(h, g, b, eps=1e-5):
    m = jnp.mean(h, axis=-1, keepdims=True)
    v = jnp.mean((h - m) ** 2, axis=-1, keepdims=True)
    return (h - m) / jnp.sqrt(v + eps) * g + b


def _gelu(h):
    return 0.5 * h * (1.0 + lax.erf(h * (2.0 ** -0.5)))


def _proj_body(x_ref, wa_ref, wb_ref, a_ref, b_ref, xn_ref):
    x = x_ref[...]
    a_ref[...] = jnp.dot(x, wa_ref[...], preferred_element_type=F32,
                         precision=_PREC)
    b_ref[...] = jnp.dot(x, wb_ref[...], preferred_element_type=F32,
                         precision=_PREC)
    xn_ref[...] = -x


def _node_proj(x, wa, wb, bn):
    n, d = x.shape
    h = wa.shape[1]
    return pl.pallas_call(
        _proj_body,
        grid=(n // bn,),
        in_specs=[pl.BlockSpec((bn, d), lambda i: (i, 0)),
                  pl.BlockSpec((d, h), lambda i: (0, 0)),
                  pl.BlockSpec((d, h), lambda i: (0, 0))],
        out_specs=[pl.BlockSpec((bn, h), lambda i: (i, 0)),
                   pl.BlockSpec((bn, h), lambda i: (i, 0)),
                   pl.BlockSpec((bn, d), lambda i: (i, 0))],
        out_shape=[jax.ShapeDtypeStruct((n, h), F32),
                   jax.ShapeDtypeStruct((n, h), F32),
                   jax.ShapeDtypeStruct((n, d), F32)],
    )(x, wa, wb)


def _sc_gather(x, xneg, aproj, bproj, row3, col3, cb=80):
    n, d = x.shape
    nw, n_chunks, _ = row3.shape
    e = nw * n_chunks * cb
    eh = aproj.shape[1]
    per_w = e // NW
    mesh = plsc.VectorSubcoreMesh(core_axis_name="c", subcore_axis_name="s")

    @functools.partial(
        pl.kernel, mesh=mesh,
        compiler_params=pltpu.CompilerParams(use_tc_tiling_on_sc=False),
        out_type=[jax.ShapeDtypeStruct((e, eh), F32),
                  jax.ShapeDtypeStruct((e, 16), F32)],
        scratch_types=[
            pltpu.VMEM((n_chunks, cb), jnp.int32),
            pltpu.VMEM((n_chunks, cb), jnp.int32),
            pltpu.VMEM((cb, d), F32),
            pltpu.VMEM((cb, d), F32),
            pltpu.VMEM((cb, eh), F32),
            pltpu.VMEM((cb, eh), F32),
            pltpu.VMEM((cb, 16), F32),
            pltpu.VMEM((cb, 16), F32),
        ] + [pltpu.SemaphoreType.DMA] * 8,
    )
    def k(x_hbm, xn_hbm, ap_hbm, bp_hbm, row_hbm, col_hbm, hp_hbm, dp_hbm,
          ir_all, ic_all, diff0, diff1, oh0, oh1, od0, od1,
          sa0, sa1, sb0, sb1, sc0, sc1, sd0, sd1):
        wid = lax.axis_index("s") * NC + lax.axis_index("c")
        pltpu.sync_copy(row_hbm.at[wid], ir_all)
        pltpu.sync_copy(col_hbm.at[wid], ic_all)

        diff = (diff0, diff1)
        oh = (oh0, oh1)
        od = (od0, od1)
        sa = (sa0, sa1)
        sb = (sb0, sb1)
        sc = (sc0, sc1)
        sd = (sd0, sd1)

        def start_ac(ci, s):
            pltpu.async_copy(x_hbm.at[ir_all.at[ci]], diff[s], sa[s])
            pltpu.async_copy(bp_hbm.at[ic_all.at[ci]], oh[s], sc[s])

        def wait_ac(s):
            pltpu.make_async_copy(x_hbm.at[ir_all.at[0]], diff[s],
                                  sa[s]).wait()
            pltpu.make_async_copy(bp_hbm.at[ic_all.at[0]], oh[s],
                                  sc[s]).wait()

        def start_bd(ci, s):
            pltpu.async_copy(xn_hbm.at[ic_all.at[ci]], diff[s], sb[s],
                             add=True)
            pltpu.async_copy(ap_hbm.at[ir_all.at[ci]], oh[s], sd[s],
                             add=True)

        def wait_bd(s):
            pltpu.make_async_copy(xn_hbm.at[ic_all.at[0]], diff[s],
                                  sb[s]).wait()
            pltpu.make_async_copy(ap_hbm.at[ir_all.at[0]], oh[s],
                                  sd[s]).wait()

        def consume(ci, s):
            wait_bd(s)
            dv = diff[s]
            ov = od[s]

            def edge(ei, _):
                acc0 = jnp.zeros((16,), F32)
                acc1 = jnp.zeros((16,), F32)
                for j in range(d // 32):
                    da = dv[ei, pl.ds(j * 32, 16)]
                    db = dv[ei, pl.ds(j * 32 + 16, 16)]
                    acc0 = acc0 + da * da
                    acc1 = acc1 + db * db
                ov[ei, :] = acc0 + acc1
                return 0

            lax.fori_loop(0, cb, edge, 0)
            base = wid * per_w + ci * cb
            pltpu.sync_copy(oh[s], hp_hbm.at[pl.ds(base, cb)])
            pltpu.sync_copy(ov, dp_hbm.at[pl.ds(base, cb)])

        start_ac(0, 0)
        wait_ac(0)
        start_bd(0, 0)
        start_ac(1, 1)

        def pair(g, _):
            for b in range(2):
                ci = 2 * g + b
                s = b
                so = 1 - b
                consume(ci, s)

                @pl.when(ci + 2 < n_chunks)
                def _():
                    start_ac(ci + 2, s)

                @pl.when(ci + 1 < n_chunks)
                def _():
                    wait_ac(so)
                    start_bd(ci + 1, so)
            return 0

        lax.fori_loop(0, (n_chunks - 1) // 2, pair, 0)
        if n_chunks % 2 == 1:
            consume(n_chunks - 1, (n_chunks - 1) % 2)

    return k(x, xneg, aproj, bproj, row3, col3)


def _edge_body(hp_ref, dp_ref, ef_ref, wef_ref, aux1_ref, we2_ref, aux2_ref,
               out_ref):
    d2 = jnp.sum(dp_ref[...], axis=1, keepdims=True)
    dist = jnp.sqrt(d2 + 1e-12)
    aux1 = aux1_ref[...]
    h = (hp_ref[...]
         + jnp.dot(ef_ref[...], wef_ref[...], preferred_element_type=F32,
                   precision=_PREC)
         + d2 * aux1[0:1] + dist * aux1[1:2] + aux1[2:3])
    h = _gelu(_ln(h, aux1[3:4], aux1[4:5]))
    aux2 = aux2_ref[...]
    h2 = jnp.dot(h, we2_ref[...], preferred_element_type=F32,
                 precision=_PREC) + aux2[0:1]
    out_ref[...] = _gelu(_ln(h2, aux2[1:2], aux2[2:3]))


def _edge_mlp(hpart, d2part, ef, wef, aux1, we2, aux2, be=4000):
    e, eh = hpart.shape
    de = ef.shape[1]
    oe = we2.shape[1]
    return pl.pallas_call(
        _edge_body,
        grid=(e // be,),
        in_specs=[pl.BlockSpec((be, eh), lambda i: (i, 0)),
                  pl.BlockSpec((be, 16), lambda i: (i, 0)),
                  pl.BlockSpec((be, de), lambda i: (i, 0)),
                  pl.BlockSpec((de, eh), lambda i: (0, 0)),
                  pl.BlockSpec((5, eh), lambda i: (0, 0)),
                  pl.BlockSpec((eh, oe), lambda i: (0, 0)),
                  pl.BlockSpec((3, oe), lambda i: (0, 0))],
        out_specs=pl.BlockSpec((be, oe), lambda i: (i, 0)),
        out_shape=jax.ShapeDtypeStruct((e, oe), F32),
    )(hpart, d2part, ef, wef, aux1, we2, aux2)


def _sc_scatter(e_arr, col3, zeros_init, cb=80):
    e, oe = e_arr.shape
    n = zeros_init.shape[0]
    nw, n_chunks, _ = col3.shape
    per_w = e // NW
    rows_per_tile = n // NS
    mesh = plsc.VectorSubcoreMesh(core_axis_name="c", subcore_axis_name="s")

    @functools.partial(
        pl.kernel, mesh=mesh,
        compiler_params=pltpu.CompilerParams(use_tc_tiling_on_sc=False),
        out_type=jax.ShapeDtypeStruct((NC * n, oe), F32),
        scratch_types=[
            pltpu.VMEM_SHARED((n, oe), F32),
            pltpu.VMEM((n_chunks, cb), jnp.int32),
            pltpu.VMEM((cb, oe), F32),
            pltpu.VMEM((cb, oe), F32),
            pltpu.SemaphoreType.DMA,
            pltpu.SemaphoreType.DMA,
        ],
    )
    def k(e_hbm, col_hbm, z_hbm, out_hbm, shared, ic_all, eb0, eb1, sl0, sl1):
        cid = lax.axis_index("c")
        sid = lax.axis_index("s")
        wid = sid * NC + cid
        eb = (eb0, eb1)
        sl = (sl0, sl1)
        pltpu.sync_copy(col_hbm.at[wid], ic_all)
        pltpu.sync_copy(z_hbm.at[pl.ds(sid * rows_per_tile, rows_per_tile)],
                        shared.at[pl.ds(sid * rows_per_tile, rows_per_tile)])
        plsc.subcore_barrier()

        def start_load(ci, s):
            base = wid * per_w + ci * cb
            pltpu.async_copy(e_hbm.at[pl.ds(base, cb)], eb[s], sl[s])

        def wait_load(s):
            pltpu.make_async_copy(e_hbm.at[pl.ds(0, cb)], eb[s], sl[s]).wait()

        def consume(ci, s):
            wait_load(s)
            pltpu.sync_copy(eb[s], shared.at[ic_all.at[ci]], add=True)

        start_load(0, 0)

        def pair(g, _):
            for b in range(2):
                ci = 2 * g + b
                s = b

                @pl.when(ci + 1 < n_chunks)
                def _():
                    start_load(ci + 1, 1 - b)

                consume(ci, s)
            return 0

        lax.fori_loop(0, n_chunks // 2, pair, 0)
        if n_chunks % 2 == 1:
            consume(n_chunks - 1, (n_chunks - 1) % 2)
        plsc.subcore_barrier()
        pltpu.sync_copy(
            shared.at[pl.ds(sid * rows_per_tile, rows_per_tile)],
            out_hbm.at[pl.ds(cid * n + sid * rows_per_tile, rows_per_tile)])

    return k(e_arr, col3, zeros_init)


def _node_body(x_ref, p0_ref, p1_ref, wx_ref, wg_ref, aux1_ref, w2_ref,
               aux2_ref, out_ref):
    agg = p0_ref[...] + p1_ref[...]
    aux1 = aux1_ref[...]
    h = (jnp.dot(x_ref[...], wx_ref[...], preferred_element_type=F32,
                 precision=_PREC)
         + jnp.dot(agg, wg_ref[...], preferred_element_type=F32,
                   precision=_PREC)
         + aux1[0:1])
    h = _gelu(_ln(h, aux1[1:2], aux1[2:3]))
    aux2 = aux2_ref[...]
    h2 = jnp.dot(h, w2_ref[...], preferred_element_type=F32,
                 precision=_PREC) + aux2[0:1]
    out_ref[...] = _gelu(_ln(h2, aux2[1:2], aux2[2:3]))


def _node_mlp(x, p0, p1, wx, wg, aux1, w2, aux2, bn=2000):
    n, d = x.shape
    oe = p0.shape[1]
    nh = wx.shape[1]
    on = w2.shape[1]
    return pl.pallas_call(
        _node_body,
        grid=(n // bn,),
        in_specs=[pl.BlockSpec((bn, d), lambda i: (i, 0)),
                  pl.BlockSpec((bn, oe), lambda i: (i, 0)),
                  pl.BlockSpec((bn, oe), lambda i: (i, 0)),
                  pl.BlockSpec((d, nh), lambda i: (0, 0)),
                  pl.BlockSpec((oe, nh), lambda i: (0, 0)),
                  pl.BlockSpec((3, nh), lambda i: (0, 0)),
                  pl.BlockSpec((nh, on), lambda i: (0, 0)),
                  pl.BlockSpec((3, on), lambda i: (0, 0))],
        out_specs=pl.BlockSpec((bn, on), lambda i: (i, 0)),
        out_shape=jax.ShapeDtypeStruct((n, on), F32),
    )(x, p0, p1, wx, wg, aux1, w2, aux2)


def kernel(node_features, edge_index, edge_features,
           W_e1, b_e1, g_e1, be_e1, W_e2, b_e2, g_e2, be_e2,
           W_n1, b_n1, g_n1, be_n1, W_n2, b_n2, g_n2, be_n2):
    n, d = node_features.shape
    e = edge_index.shape[1]
    oe = W_e2.shape[1]

    ws, wt, wd = W_e1[:d], W_e1[d:2 * d], W_e1[2 * d:3 * d]
    w_d2, w_dist = W_e1[3 * d], W_e1[3 * d + 1]
    wef = W_e1[3 * d + 2:]
    wa = ws + wd
    wb = wt - wd
    row = edge_index[0]
    col = edge_index[1]

    cb = 80
    row3 = row.reshape(NW, e // (NW * cb), cb)
    col3 = col.reshape(NW, e // (NW * cb), cb)
    aproj, bproj, xneg = _node_proj(node_features, wa, wb, bn=2000)
    hpart, d2part = _sc_gather(node_features, xneg, aproj, bproj, row3, col3,
                               cb=cb)

    aux_e1 = jnp.stack([w_d2, w_dist, b_e1, g_e1, be_e1])
    aux_e2 = jnp.stack([b_e2, g_e2, be_e2])
    e_out = _edge_mlp(hpart, d2part, edge_features, wef, aux_e1, W_e2, aux_e2)

    parts = _sc_scatter(e_out, col3, jnp.zeros((n, oe), F32), cb=cb)
    p0, p1 = parts[:n], parts[n:]

    aux_n1 = jnp.stack([b_n1, g_n1, be_n1])
    aux_n2 = jnp.stack([b_n2, g_n2, be_n2])
    n_out = _node_mlp(node_features, p0, p1, W_n1[:d], W_n1[d:],
                      aux_n1, W_n2, aux_n2)
    return (n_out, e_out)


# speedup vs baseline: 2.8982x; 1.2833x over previous
import functools

import jax
import jax.numpy as jnp
from jax import lax
from jax.experimental import pallas as pl
from jax.experimental.pallas import tpu as pltpu
from jax.experimental.pallas import tpu_sc as plsc

F32 = jnp.float32
_PREC = lax.Precision.HIGHEST

NC = 2
NS = 16
NW = NC * NS


def _ln(h, g, b, eps=1e-5):
    m = jnp.mean(h, axis=-1, keepdims=True)
    v = jnp.mean((h - m) ** 2, axis=-1, keepdims=True)
    return (h - m) / jnp.sqrt(v + eps) * g + b


def _gelu(h):
    return 0.5 * h * (1.0 + lax.erf(h * (2.0 ** -0.5)))


def _proj_body(x_ref, wa_ref, wb_ref, a_ref, b_ref, xn_ref):
    x = x_ref[...]
    a_ref[...] = jnp.dot(x, wa_ref[...], preferred_element_type=F32,
                         precision=_PREC)
    b_ref[...] = jnp.dot(x, wb_ref[...], preferred_element_type=F32,
                         precision=_PREC)
    xn_ref[...] = -x


def _node_proj(x, wa, wb, bn):
    n, d = x.shape
    h = wa.shape[1]
    return pl.pallas_call(
        _proj_body,
        grid=(n // bn,),
        in_specs=[pl.BlockSpec((bn, d), lambda i: (i, 0)),
                  pl.BlockSpec((d, h), lambda i: (0, 0)),
                  pl.BlockSpec((d, h), lambda i: (0, 0))],
        out_specs=[pl.BlockSpec((bn, h), lambda i: (i, 0)),
                   pl.BlockSpec((bn, h), lambda i: (i, 0)),
                   pl.BlockSpec((bn, d), lambda i: (i, 0))],
        out_shape=[jax.ShapeDtypeStruct((n, h), F32),
                   jax.ShapeDtypeStruct((n, h), F32),
                   jax.ShapeDtypeStruct((n, d), F32)],
    )(x, wa, wb)


def _sc_gather(x, xneg, aproj, bproj, row, col, cb=80):
    n, d = x.shape
    e = row.shape[0]
    eh = aproj.shape[1]
    per_w = e // NW
    n_chunks = per_w // cb
    mesh = plsc.VectorSubcoreMesh(core_axis_name="c", subcore_axis_name="s")

    @functools.partial(
        pl.kernel, mesh=mesh,
        compiler_params=pltpu.CompilerParams(use_tc_tiling_on_sc=False),
        out_type=[jax.ShapeDtypeStruct((e, eh), F32),
                  jax.ShapeDtypeStruct((e, 16), F32)],
        scratch_types=[
            pltpu.VMEM((per_w,), jnp.int32),
            pltpu.VMEM((per_w,), jnp.int32),
            pltpu.VMEM((cb, d), F32),
            pltpu.VMEM((cb, d), F32),
            pltpu.VMEM((cb, eh), F32),
            pltpu.VMEM((cb, eh), F32),
            pltpu.VMEM((cb, 16), F32),
            pltpu.VMEM((cb, 16), F32),
        ] + [pltpu.SemaphoreType.DMA] * 8,
    )
    def k(x_hbm, xn_hbm, ap_hbm, bp_hbm, row_hbm, col_hbm, hp_hbm, dp_hbm,
          ir_all, ic_all, diff0, diff1, oh0, oh1, od0, od1,
          sa0, sa1, sb0, sb1, sc0, sc1, sd0, sd1):
        wid = lax.axis_index("s") * NC + lax.axis_index("c")
        pltpu.sync_copy(row_hbm.at[pl.ds(wid * per_w, per_w)], ir_all)
        pltpu.sync_copy(col_hbm.at[pl.ds(wid * per_w, per_w)], ic_all)

        diff = (diff0, diff1)
        oh = (oh0, oh1)
        od = (od0, od1)
        sa = (sa0, sa1)
        sb = (sb0, sb1)
        sc = (sc0, sc1)
        sd = (sd0, sd1)

        def start_ac(ci, s):
            ir = ir_all.at[pl.ds(ci * cb, cb)]
            ic = ic_all.at[pl.ds(ci * cb, cb)]
            pltpu.async_copy(x_hbm.at[ir], diff[s], sa[s])
            pltpu.async_copy(bp_hbm.at[ic], oh[s], sc[s])

        def wait_ac(s):
            ir0 = ir_all.at[pl.ds(0, cb)]
            pltpu.make_async_copy(x_hbm.at[ir0], diff[s], sa[s]).wait()
            pltpu.make_async_copy(bp_hbm.at[ir0], oh[s], sc[s]).wait()

        def start_bd(ci, s):
            ir = ir_all.at[pl.ds(ci * cb, cb)]
            ic = ic_all.at[pl.ds(ci * cb, cb)]
            pltpu.async_copy(xn_hbm.at[ic], diff[s], sb[s], add=True)
            pltpu.async_copy(ap_hbm.at[ir], oh[s], sd[s], add=True)

        def wait_bd(s):
            ir0 = ir_all.at[pl.ds(0, cb)]
            pltpu.make_async_copy(xn_hbm.at[ir0], diff[s], sb[s]).wait()
            pltpu.make_async_copy(ap_hbm.at[ir0], oh[s], sd[s]).wait()

        def consume(ci, s):
            wait_bd(s)
            dv = diff[s]
            ov = od[s]

            def edge(ei, _):
                acc0 = jnp.zeros((16,), F32)
                acc1 = jnp.zeros((16,), F32)
                for j in range(d // 32):
                    da = dv[ei, pl.ds(j * 32, 16)]
                    db = dv[ei, pl.ds(j * 32 + 16, 16)]
                    acc0 = acc0 + da * da
                    acc1 = acc1 + db * db
                ov[ei, :] = acc0 + acc1
                return 0

            lax.fori_loop(0, cb, edge, 0)
            base = wid * per_w + ci * cb
            pltpu.sync_copy(oh[s], hp_hbm.at[pl.ds(base, cb)])
            pltpu.sync_copy(ov, dp_hbm.at[pl.ds(base, cb)])

        start_ac(0, 0)
        wait_ac(0)
        start_bd(0, 0)
        start_ac(1, 1)

        def pair(g, _):
            for b in range(2):
                ci = 2 * g + b
                s = b
                so = 1 - b
                consume(ci, s)

                @pl.when(ci + 2 < n_chunks)
                def _():
                    start_ac(ci + 2, s)

                @pl.when(ci + 1 < n_chunks)
                def _():
                    wait_ac(so)
                    start_bd(ci + 1, so)
            return 0

        lax.fori_loop(0, (n_chunks - 1) // 2, pair, 0)
        if n_chunks % 2 == 1:
            consume(n_chunks - 1, (n_chunks - 1) % 2)

    return k(x, xneg, aproj, bproj, row, col)


def _edge_body(hp_ref, dp_ref, ef_ref, wef_ref, aux1_ref, we2_ref, aux2_ref,
               out_ref):
    d2 = jnp.sum(dp_ref[...], axis=1, keepdims=True)
    dist = jnp.sqrt(d2 + 1e-12)
    aux1 = aux1_ref[...]
    h = (hp_ref[...]
         + jnp.dot(ef_ref[...], wef_ref[...], preferred_element_type=F32)
         + d2 * aux1[0:1] + dist * aux1[1:2] + aux1[2:3])
    h = _gelu(_ln(h, aux1[3:4], aux1[4:5]))
    aux2 = aux2_ref[...]
    h2 = jnp.dot(h, we2_ref[...], preferred_element_type=F32) + aux2[0:1]
    out_ref[...] = _gelu(_ln(h2, aux2[1:2], aux2[2:3]))


def _edge_mlp(hpart, d2part, ef, wef, aux1, we2, aux2, be=4000):
    e, eh = hpart.shape
    de = ef.shape[1]
    oe = we2.shape[1]
    return pl.pallas_call(
        _edge_body,
        grid=(e // be,),
        in_specs=[pl.BlockSpec((be, eh), lambda i: (i, 0)),
                  pl.BlockSpec((be, 16), lambda i: (i, 0)),
                  pl.BlockSpec((be, de), lambda i: (i, 0)),
                  pl.BlockSpec((de, eh), lambda i: (0, 0)),
                  pl.BlockSpec((5, eh), lambda i: (0, 0)),
                  pl.BlockSpec((eh, oe), lambda i: (0, 0)),
                  pl.BlockSpec((3, oe), lambda i: (0, 0))],
        out_specs=pl.BlockSpec((be, oe), lambda i: (i, 0)),
        out_shape=jax.ShapeDtypeStruct((e, oe), F32),
    )(hpart, d2part, ef, wef, aux1, we2, aux2)


def _sc_scatter(e_arr, col, zeros_init, cb=80):
    e, oe = e_arr.shape
    n = zeros_init.shape[0]
    per_w = e // NW
    n_chunks = per_w // cb
    rows_per_tile = n // NS
    mesh = plsc.VectorSubcoreMesh(core_axis_name="c", subcore_axis_name="s")

    @functools.partial(
        pl.kernel, mesh=mesh,
        compiler_params=pltpu.CompilerParams(use_tc_tiling_on_sc=False),
        out_type=jax.ShapeDtypeStruct((NC * n, oe), F32),
        scratch_types=[
            pltpu.VMEM_SHARED((n, oe), F32),
            pltpu.VMEM((per_w,), jnp.int32),
            pltpu.VMEM((cb, oe), F32),
            pltpu.VMEM((cb, oe), F32),
            pltpu.SemaphoreType.DMA,
            pltpu.SemaphoreType.DMA,
        ],
    )
    def k(e_hbm, col_hbm, z_hbm, out_hbm, shared, ic_all, eb0, eb1, sl0, sl1):
        cid = lax.axis_index("c")
        sid = lax.axis_index("s")
        wid = sid * NC + cid
        eb = (eb0, eb1)
        sl = (sl0, sl1)
        pltpu.sync_copy(col_hbm.at[pl.ds(wid * per_w, per_w)], ic_all)
        pltpu.sync_copy(z_hbm.at[pl.ds(sid * rows_per_tile, rows_per_tile)],
                        shared.at[pl.ds(sid * rows_per_tile, rows_per_tile)])
        plsc.subcore_barrier()

        def start_load(ci, s):
            base = wid * per_w + ci * cb
            pltpu.async_copy(e_hbm.at[pl.ds(base, cb)], eb[s], sl[s])

        def wait_load(s):
            pltpu.make_async_copy(e_hbm.at[pl.ds(0, cb)], eb[s], sl[s]).wait()

        def consume(ci, s):
            wait_load(s)
            pltpu.sync_copy(eb[s], shared.at[ic_all.at[pl.ds(ci * cb, cb)]],
                            add=True)

        start_load(0, 0)

        def pair(g, _):
            for b in range(2):
                ci = 2 * g + b
                s = b

                @pl.when(ci + 1 < n_chunks)
                def _():
                    start_load(ci + 1, 1 - b)

                consume(ci, s)
            return 0

        lax.fori_loop(0, n_chunks // 2, pair, 0)
        if n_chunks % 2 == 1:
            consume(n_chunks - 1, (n_chunks - 1) % 2)
        plsc.subcore_barrier()
        pltpu.sync_copy(
            shared.at[pl.ds(sid * rows_per_tile, rows_per_tile)],
            out_hbm.at[pl.ds(cid * n + sid * rows_per_tile, rows_per_tile)])

    return k(e_arr, col, zeros_init)


def _node_body(x_ref, p0_ref, p1_ref, wx_ref, wg_ref, aux1_ref, w2_ref,
               aux2_ref, out_ref):
    agg = p0_ref[...] + p1_ref[...]
    aux1 = aux1_ref[...]
    h = (jnp.dot(x_ref[...], wx_ref[...], preferred_element_type=F32,
                 precision=_PREC)
         + jnp.dot(agg, wg_ref[...], preferred_element_type=F32,
                   precision=_PREC)
         + aux1[0:1])
    h = _gelu(_ln(h, aux1[1:2], aux1[2:3]))
    aux2 = aux2_ref[...]
    h2 = jnp.dot(h, w2_ref[...], preferred_element_type=F32,
                 precision=_PREC) + aux2[0:1]
    out_ref[...] = _gelu(_ln(h2, aux2[1:2], aux2[2:3]))


def _node_mlp(x, p0, p1, wx, wg, aux1, w2, aux2, bn=2000):
    n, d = x.shape
    oe = p0.shape[1]
    nh = wx.shape[1]
    on = w2.shape[1]
    return pl.pallas_call(
        _node_body,
        grid=(n // bn,),
        in_specs=[pl.BlockSpec((bn, d), lambda i: (i, 0)),
                  pl.BlockSpec((bn, oe), lambda i: (i, 0)),
                  pl.BlockSpec((bn, oe), lambda i: (i, 0)),
                  pl.BlockSpec((d, nh), lambda i: (0, 0)),
                  pl.BlockSpec((oe, nh), lambda i: (0, 0)),
                  pl.BlockSpec((3, nh), lambda i: (0, 0)),
                  pl.BlockSpec((nh, on), lambda i: (0, 0)),
                  pl.BlockSpec((3, on), lambda i: (0, 0))],
        out_specs=pl.BlockSpec((bn, on), lambda i: (i, 0)),
        out_shape=jax.ShapeDtypeStruct((n, on), F32),
    )(x, p0, p1, wx, wg, aux1, w2, aux2)


def kernel(node_features, edge_index, edge_features,
           W_e1, b_e1, g_e1, be_e1, W_e2, b_e2, g_e2, be_e2,
           W_n1, b_n1, g_n1, be_n1, W_n2, b_n2, g_n2, be_n2):
    n, d = node_features.shape
    e = edge_index.shape[1]
    oe = W_e2.shape[1]

    ws, wt, wd = W_e1[:d], W_e1[d:2 * d], W_e1[2 * d:3 * d]
    w_d2, w_dist = W_e1[3 * d], W_e1[3 * d + 1]
    wef = W_e1[3 * d + 2:]
    wa = ws + wd
    wb = wt - wd
    row = edge_index[0]
    col = edge_index[1]

    aproj, bproj, xneg = _node_proj(node_features, wa, wb, bn=2000)
    hpart, d2part = _sc_gather(node_features, xneg, aproj, bproj, row, col)

    aux_e1 = jnp.stack([w_d2, w_dist, b_e1, g_e1, be_e1])
    aux_e2 = jnp.stack([b_e2, g_e2, be_e2])
    e_out = _edge_mlp(hpart, d2part, edge_features, wef, aux_e1, W_e2, aux_e2)

    parts = _sc_scatter(e_out, col, jnp.zeros((n, oe), F32))
    p0, p1 = parts[:n], parts[n:]

    aux_n1 = jnp.stack([b_n1, g_n1, be_n1])
    aux_n2 = jnp.stack([b_n2, g_n2, be_n2])
    n_out = _node_mlp(node_features, p0, p1, W_n1[:d], W_n1[d:],
                      aux_n1, W_n2, aux_n2)
    return (n_out, e_out)


# speedup vs baseline: 3.3160x; 1.1442x over previous
import functools

import jax
import jax.numpy as jnp
from jax import lax
from jax.experimental import pallas as pl
from jax.experimental.pallas import tpu as pltpu
from jax.experimental.pallas import tpu_sc as plsc

F32 = jnp.float32
_PREC = lax.Precision.HIGHEST

NC = 2
NS = 16
NW = NC * NS


def _ln(h, g, b, eps=1e-5):
    m = jnp.mean(h, axis=-1, keepdims=True)
    v = jnp.mean((h - m) ** 2, axis=-1, keepdims=True)
    return (h - m) / jnp.sqrt(v + eps) * g + b


def _gelu(h):
    return 0.5 * h * (1.0 + lax.erf(h * (2.0 ** -0.5)))


def _proj_body(x_ref, wa_ref, wb_ref, a_ref, b_ref, xn_ref):
    x = x_ref[...]
    a_ref[...] = jnp.dot(x, wa_ref[...], preferred_element_type=F32,
                         precision=_PREC)
    b_ref[...] = jnp.dot(x, wb_ref[...], preferred_element_type=F32,
                         precision=_PREC)
    xn_ref[...] = -x


def _node_proj(x, wa, wb, bn):
    n, d = x.shape
    h = wa.shape[1]
    return pl.pallas_call(
        _proj_body,
        grid=(n // bn,),
        in_specs=[pl.BlockSpec((bn, d), lambda i: (i, 0)),
                  pl.BlockSpec((d, h), lambda i: (0, 0)),
                  pl.BlockSpec((d, h), lambda i: (0, 0))],
        out_specs=[pl.BlockSpec((bn, h), lambda i: (i, 0)),
                   pl.BlockSpec((bn, h), lambda i: (i, 0)),
                   pl.BlockSpec((bn, d), lambda i: (i, 0))],
        out_shape=[jax.ShapeDtypeStruct((n, h), F32),
                   jax.ShapeDtypeStruct((n, h), F32),
                   jax.ShapeDtypeStruct((n, d), F32)],
    )(x, wa, wb)


def _sc_gather(x, xneg, aproj, bproj, row, col, cb=80):
    n, d = x.shape
    e = row.shape[0]
    eh = aproj.shape[1]
    per_w = e // NW
    n_chunks = per_w // cb
    mesh = plsc.VectorSubcoreMesh(core_axis_name="c", subcore_axis_name="s")

    @functools.partial(
        pl.kernel, mesh=mesh,
        compiler_params=pltpu.CompilerParams(use_tc_tiling_on_sc=False),
        out_type=jax.ShapeDtypeStruct((e, 128), F32),
        scratch_types=[
            pltpu.VMEM((per_w,), jnp.int32),
            pltpu.VMEM((per_w,), jnp.int32),
            pltpu.VMEM((cb, d), F32),
            pltpu.VMEM((cb, d), F32),
            pltpu.VMEM((cb, eh), F32),
            pltpu.VMEM((cb, eh), F32),
            pltpu.VMEM((cb, 128), F32),
            pltpu.VMEM((cb, 128), F32),
        ] + [pltpu.SemaphoreType.DMA] * 8,
    )
    def k(x_hbm, xn_hbm, ap_hbm, bp_hbm, row_hbm, col_hbm, hp_hbm,
          ir_all, ic_all, diff0, diff1, oh0, oh1, od0, od1,
          sa0, sa1, sb0, sb1, sc0, sc1, sd0, sd1):
        wid = lax.axis_index("s") * NC + lax.axis_index("c")
        pltpu.sync_copy(row_hbm.at[pl.ds(wid * per_w, per_w)], ir_all)
        pltpu.sync_copy(col_hbm.at[pl.ds(wid * per_w, per_w)], ic_all)

        diff = (diff0, diff1)
        oh = (oh0, oh1)
        od = (od0, od1)
        sa = (sa0, sa1)
        sb = (sb0, sb1)
        sc = (sc0, sc1)
        sd = (sd0, sd1)

        def start_ac(ci, s):
            ir = ir_all.at[pl.ds(ci * cb, cb)]
            ic = ic_all.at[pl.ds(ci * cb, cb)]
            pltpu.async_copy(x_hbm.at[ir], diff[s], sa[s])
            pltpu.async_copy(bp_hbm.at[ic], oh[s], sc[s])

        def wait_ac(s):
            ir0 = ir_all.at[pl.ds(0, cb)]
            pltpu.make_async_copy(x_hbm.at[ir0], diff[s], sa[s]).wait()
            pltpu.make_async_copy(bp_hbm.at[ir0], oh[s], sc[s]).wait()

        def start_bd(ci, s):
            ir = ir_all.at[pl.ds(ci * cb, cb)]
            ic = ic_all.at[pl.ds(ci * cb, cb)]
            pltpu.async_copy(xn_hbm.at[ic], diff[s], sb[s], add=True)
            pltpu.async_copy(ap_hbm.at[ir], oh[s], sd[s], add=True)

        def wait_bd(s):
            ir0 = ir_all.at[pl.ds(0, cb)]
            pltpu.make_async_copy(xn_hbm.at[ir0], diff[s], sb[s]).wait()
            pltpu.make_async_copy(ap_hbm.at[ir0], oh[s], sd[s]).wait()

        def consume(ci, s):
            wait_bd(s)
            dv = diff[s]
            hv = oh[s]
            ov = od[s]

            def edge(ei, _):
                acc0 = jnp.zeros((16,), F32)
                acc1 = jnp.zeros((16,), F32)
                for j in range(d // 32):
                    da = dv[ei, pl.ds(j * 32, 16)]
                    db = dv[ei, pl.ds(j * 32 + 16, 16)]
                    acc0 = acc0 + da * da
                    acc1 = acc1 + db * db
                ov[ei, pl.ds(48, 16)] = acc0 + acc1
                for j in range(eh // 16):
                    ov[ei, pl.ds(j * 16, 16)] = hv[ei, pl.ds(j * 16, 16)]
                return 0

            lax.fori_loop(0, cb, edge, 0)
            base = wid * per_w + ci * cb
            pltpu.sync_copy(ov, hp_hbm.at[pl.ds(base, cb)])

        start_ac(0, 0)
        wait_ac(0)
        start_bd(0, 0)
        start_ac(1, 1)

        def pair(g, _):
            for b in range(2):
                ci = 2 * g + b
                s = b
                so = 1 - b
                consume(ci, s)

                @pl.when(ci + 2 < n_chunks)
                def _():
                    start_ac(ci + 2, s)

                @pl.when(ci + 1 < n_chunks)
                def _():
                    wait_ac(so)
                    start_bd(ci + 1, so)
            return 0

        lax.fori_loop(0, (n_chunks - 1) // 2, pair, 0)
        if n_chunks % 2 == 1:
            consume(n_chunks - 1, (n_chunks - 1) % 2)

    return k(x, xneg, aproj, bproj, row, col)


def _edge_body(hp_ref, ef_ref, wef_ref, aux1_ref, we2_ref, aux2_ref,
               out_ref):
    blk = hp_ref[...]
    d2 = jnp.sum(blk[:, 48:64], axis=1, keepdims=True)
    dist = jnp.sqrt(d2 + 1e-12)
    aux1 = aux1_ref[...]
    h = (blk[:, 0:48]
         + jnp.dot(ef_ref[...], wef_ref[...], preferred_element_type=F32)
         + d2 * aux1[0:1] + dist * aux1[1:2] + aux1[2:3])
    h = _gelu(_ln(h, aux1[3:4], aux1[4:5]))
    aux2 = aux2_ref[...]
    h2 = jnp.dot(h, we2_ref[...], preferred_element_type=F32) + aux2[0:1]
    out_ref[...] = _gelu(_ln(h2, aux2[1:2], aux2[2:3]))


def _edge_mlp(hpart, ef, wef, aux1, we2, aux2, be=4000):
    e = hpart.shape[0]
    eh = wef.shape[1]
    de = ef.shape[1]
    oe = we2.shape[1]
    return pl.pallas_call(
        _edge_body,
        grid=(e // be,),
        in_specs=[pl.BlockSpec((be, 128), lambda i: (i, 0)),
                  pl.BlockSpec((be, de), lambda i: (i, 0)),
                  pl.BlockSpec((de, eh), lambda i: (0, 0)),
                  pl.BlockSpec((5, eh), lambda i: (0, 0)),
                  pl.BlockSpec((eh, oe), lambda i: (0, 0)),
                  pl.BlockSpec((3, oe), lambda i: (0, 0))],
        out_specs=pl.BlockSpec((be, oe), lambda i: (i, 0)),
        out_shape=jax.ShapeDtypeStruct((e, oe), F32),
    )(hpart, ef, wef, aux1, we2, aux2)


def _sc_scatter(e_arr, col, zeros_init, cb=80):
    e, oe = e_arr.shape
    n = zeros_init.shape[0]
    per_w = e // NW
    n_chunks = per_w // cb
    rows_per_tile = n // NS
    mesh = plsc.VectorSubcoreMesh(core_axis_name="c", subcore_axis_name="s")

    @functools.partial(
        pl.kernel, mesh=mesh,
        compiler_params=pltpu.CompilerParams(use_tc_tiling_on_sc=False),
        out_type=jax.ShapeDtypeStruct((NC * n, oe), F32),
        scratch_types=[
            pltpu.VMEM_SHARED((n, oe), F32),
            pltpu.VMEM((per_w,), jnp.int32),
            pltpu.VMEM((cb, oe), F32),
            pltpu.VMEM((cb, oe), F32),
            pltpu.SemaphoreType.DMA,
            pltpu.SemaphoreType.DMA,
        ],
    )
    def k(e_hbm, col_hbm, z_hbm, out_hbm, shared, ic_all, eb0, eb1, sl0, sl1):
        cid = lax.axis_index("c")
        sid = lax.axis_index("s")
        wid = sid * NC + cid
        eb = (eb0, eb1)
        sl = (sl0, sl1)
        pltpu.sync_copy(col_hbm.at[pl.ds(wid * per_w, per_w)], ic_all)
        pltpu.sync_copy(z_hbm.at[pl.ds(sid * rows_per_tile, rows_per_tile)],
                        shared.at[pl.ds(sid * rows_per_tile, rows_per_tile)])
        plsc.subcore_barrier()

        def start_load(ci, s):
            base = wid * per_w + ci * cb
            pltpu.async_copy(e_hbm.at[pl.ds(base, cb)], eb[s], sl[s])

        def wait_load(s):
            pltpu.make_async_copy(e_hbm.at[pl.ds(0, cb)], eb[s], sl[s]).wait()

        def consume(ci, s):
            wait_load(s)
            pltpu.sync_copy(eb[s], shared.at[ic_all.at[pl.ds(ci * cb, cb)]],
                            add=True)

        start_load(0, 0)

        def pair(g, _):
            for b in range(2):
                ci = 2 * g + b
                s = b

                @pl.when(ci + 1 < n_chunks)
                def _():
                    start_load(ci + 1, 1 - b)

                consume(ci, s)
            return 0

        lax.fori_loop(0, n_chunks // 2, pair, 0)
        if n_chunks % 2 == 1:
            consume(n_chunks - 1, (n_chunks - 1) % 2)
        plsc.subcore_barrier()
        pltpu.sync_copy(
            shared.at[pl.ds(sid * rows_per_tile, rows_per_tile)],
            out_hbm.at[pl.ds(cid * n + sid * rows_per_tile, rows_per_tile)])

    return k(e_arr, col, zeros_init)


def _node_body(x_ref, p0_ref, p1_ref, wx_ref, wg_ref, aux1_ref, w2_ref,
               aux2_ref, out_ref):
    agg = p0_ref[...] + p1_ref[...]
    aux1 = aux1_ref[...]
    h = (jnp.dot(x_ref[...], wx_ref[...], preferred_element_type=F32,
                 precision=_PREC)
         + jnp.dot(agg, wg_ref[...], preferred_element_type=F32,
                   precision=_PREC)
         + aux1[0:1])
    h = _gelu(_ln(h, aux1[1:2], aux1[2:3]))
    aux2 = aux2_ref[...]
    h2 = jnp.dot(h, w2_ref[...], preferred_element_type=F32,
                 precision=_PREC) + aux2[0:1]
    out_ref[...] = _gelu(_ln(h2, aux2[1:2], aux2[2:3]))


def _node_mlp(x, p0, p1, wx, wg, aux1, w2, aux2, bn=2000):
    n, d = x.shape
    oe = p0.shape[1]
    nh = wx.shape[1]
    on = w2.shape[1]
    return pl.pallas_call(
        _node_body,
        grid=(n // bn,),
        in_specs=[pl.BlockSpec((bn, d), lambda i: (i, 0)),
                  pl.BlockSpec((bn, oe), lambda i: (i, 0)),
                  pl.BlockSpec((bn, oe), lambda i: (i, 0)),
                  pl.BlockSpec((d, nh), lambda i: (0, 0)),
                  pl.BlockSpec((oe, nh), lambda i: (0, 0)),
                  pl.BlockSpec((3, nh), lambda i: (0, 0)),
                  pl.BlockSpec((nh, on), lambda i: (0, 0)),
                  pl.BlockSpec((3, on), lambda i: (0, 0))],
        out_specs=pl.BlockSpec((bn, on), lambda i: (i, 0)),
        out_shape=jax.ShapeDtypeStruct((n, on), F32),
    )(x, p0, p1, wx, wg, aux1, w2, aux2)


def kernel(node_features, edge_index, edge_features,
           W_e1, b_e1, g_e1, be_e1, W_e2, b_e2, g_e2, be_e2,
           W_n1, b_n1, g_n1, be_n1, W_n2, b_n2, g_n2, be_n2):
    n, d = node_features.shape
    e = edge_index.shape[1]
    oe = W_e2.shape[1]

    ws, wt, wd = W_e1[:d], W_e1[d:2 * d], W_e1[2 * d:3 * d]
    w_d2, w_dist = W_e1[3 * d], W_e1[3 * d + 1]
    wef = W_e1[3 * d + 2:]
    wa = ws + wd
    wb = wt - wd
    row = edge_index[0]
    col = edge_index[1]

    aproj, bproj, xneg = _node_proj(node_features, wa, wb, bn=2000)
    hpart = _sc_gather(node_features, xneg, aproj, bproj, row, col)

    aux_e1 = jnp.stack([w_d2, w_dist, b_e1, g_e1, be_e1])
    aux_e2 = jnp.stack([b_e2, g_e2, be_e2])
    e_out = _edge_mlp(hpart, edge_features, wef, aux_e1, W_e2, aux_e2)

    parts = _sc_scatter(e_out, col, jnp.zeros((n, oe), F32))
    p0, p1 = parts[:n], parts[n:]

    aux_n1 = jnp.stack([b_n1, g_n1, be_n1])
    aux_n2 = jnp.stack([b_n2, g_n2, be_n2])
    n_out = _node_mlp(node_features, p0, p1, W_n1[:d], W_n1[d:],
                      aux_n1, W_n2, aux_n2)
    return (n_out, e_out)


# speedup vs baseline: 3.4782x; 1.0489x over previous
import functools

import jax
import jax.numpy as jnp
from jax import lax
from jax.experimental import pallas as pl
from jax.experimental.pallas import tpu as pltpu
from jax.experimental.pallas import tpu_sc as plsc

F32 = jnp.float32
_PREC = lax.Precision.HIGHEST

NC = 2
NS = 16
NW = NC * NS


def _ln(h, g, b, eps=1e-5):
    m = jnp.mean(h, axis=-1, keepdims=True)
    v = jnp.mean((h - m) ** 2, axis=-1, keepdims=True)
    return (h - m) / jnp.sqrt(v + eps) * g + b


def _gelu(h):
    return 0.5 * h * (1.0 + lax.erf(h * (2.0 ** -0.5)))


def _proj_body(x_ref, wa_ref, wb_ref, a_ref, b_ref, xn_ref):
    x = x_ref[...]
    a_ref[...] = jnp.dot(x, wa_ref[...], preferred_element_type=F32,
                         precision=_PREC)
    b_ref[...] = jnp.dot(x, wb_ref[...], preferred_element_type=F32,
                         precision=_PREC)
    xn_ref[...] = -x


def _node_proj(x, wa, wb, bn):
    n, d = x.shape
    h = wa.shape[1]
    return pl.pallas_call(
        _proj_body,
        grid=(n // bn,),
        in_specs=[pl.BlockSpec((bn, d), lambda i: (i, 0)),
                  pl.BlockSpec((d, h), lambda i: (0, 0)),
                  pl.BlockSpec((d, h), lambda i: (0, 0))],
        out_specs=[pl.BlockSpec((bn, h), lambda i: (i, 0)),
                   pl.BlockSpec((bn, h), lambda i: (i, 0)),
                   pl.BlockSpec((bn, d), lambda i: (i, 0))],
        out_shape=[jax.ShapeDtypeStruct((n, h), F32),
                   jax.ShapeDtypeStruct((n, h), F32),
                   jax.ShapeDtypeStruct((n, d), F32)],
    )(x, wa, wb)


def _sc_gather(x, xneg, aproj, bproj, row, col, cb=80):
    n, d = x.shape
    e = row.shape[0]
    eh = aproj.shape[1]
    per_w = e // NW
    n_chunks = per_w // cb
    mesh = plsc.VectorSubcoreMesh(core_axis_name="c", subcore_axis_name="s")

    @functools.partial(
        pl.kernel, mesh=mesh,
        compiler_params=pltpu.CompilerParams(use_tc_tiling_on_sc=False),
        out_type=jax.ShapeDtypeStruct((e, 128), F32),
        scratch_types=[
            pltpu.VMEM((per_w,), jnp.int32),
            pltpu.VMEM((per_w,), jnp.int32),
            pltpu.VMEM((cb, d), F32),
            pltpu.VMEM((cb, d), F32),
            pltpu.VMEM((cb, eh), F32),
            pltpu.VMEM((cb, eh), F32),
            pltpu.VMEM((cb, 128), F32),
            pltpu.VMEM((cb, 128), F32),
        ] + [pltpu.SemaphoreType.DMA] * 8,
    )
    def k(x_hbm, xn_hbm, ap_hbm, bp_hbm, row_hbm, col_hbm, hp_hbm,
          ir_all, ic_all, diff0, diff1, oh0, oh1, od0, od1,
          sa0, sa1, sb0, sb1, sc0, sc1, sd0, sd1):
        wid = lax.axis_index("s") * NC + lax.axis_index("c")
        pltpu.sync_copy(row_hbm.at[pl.ds(wid * per_w, per_w)], ir_all)
        pltpu.sync_copy(col_hbm.at[pl.ds(wid * per_w, per_w)], ic_all)

        diff = (diff0, diff1)
        oh = (oh0, oh1)
        od = (od0, od1)
        sa = (sa0, sa1)
        sb = (sb0, sb1)
        sc = (sc0, sc1)
        sd = (sd0, sd1)

        def start_ac(ci, s):
            ir = ir_all.at[pl.ds(ci * cb, cb)]
            ic = ic_all.at[pl.ds(ci * cb, cb)]
            pltpu.async_copy(x_hbm.at[ir], diff[s], sa[s])
            pltpu.async_copy(bp_hbm.at[ic], oh[s], sc[s])

        def wait_ac(s):
            ir0 = ir_all.at[pl.ds(0, cb)]
            pltpu.make_async_copy(x_hbm.at[ir0], diff[s], sa[s]).wait()
            pltpu.make_async_copy(bp_hbm.at[ir0], oh[s], sc[s]).wait()

        def start_bd(ci, s):
            ir = ir_all.at[pl.ds(ci * cb, cb)]
            ic = ic_all.at[pl.ds(ci * cb, cb)]
            pltpu.async_copy(xn_hbm.at[ic], diff[s], sb[s], add=True)
            pltpu.async_copy(ap_hbm.at[ir], oh[s], sd[s], add=True)

        def wait_bd(s):
            ir0 = ir_all.at[pl.ds(0, cb)]
            pltpu.make_async_copy(xn_hbm.at[ir0], diff[s], sb[s]).wait()
            pltpu.make_async_copy(ap_hbm.at[ir0], oh[s], sd[s]).wait()

        def consume(ci, s):
            wait_bd(s)
            dv = diff[s]
            hv = oh[s]
            ov = od[s]

            def edge(ei, _):
                acc0 = jnp.zeros((16,), F32)
                acc1 = jnp.zeros((16,), F32)
                for j in range(d // 32):
                    da = dv[ei, pl.ds(j * 32, 16)]
                    db = dv[ei, pl.ds(j * 32 + 16, 16)]
                    acc0 = acc0 + da * da
                    acc1 = acc1 + db * db
                ov[ei, pl.ds(48, 16)] = acc0 + acc1
                for j in range(eh // 16):
                    ov[ei, pl.ds(j * 16, 16)] = hv[ei, pl.ds(j * 16, 16)]
                return 0

            lax.fori_loop(0, cb, edge, 0)
            base = wid * per_w + ci * cb
            pltpu.sync_copy(ov, hp_hbm.at[pl.ds(base, cb)])

        start_ac(0, 0)
        wait_ac(0)
        start_bd(0, 0)
        start_ac(1, 1)

        def pair(g, _):
            for b in range(2):
                ci = 2 * g + b
                s = b
                so = 1 - b
                consume(ci, s)

                @pl.when(ci + 2 < n_chunks)
                def _():
                    start_ac(ci + 2, s)

                @pl.when(ci + 1 < n_chunks)
                def _():
                    wait_ac(so)
                    start_bd(ci + 1, so)
            return 0

        lax.fori_loop(0, (n_chunks - 1) // 2, pair, 0)
        if n_chunks % 2 == 1:
            consume(n_chunks - 1, (n_chunks - 1) % 2)

    return k(x, xneg, aproj, bproj, row, col)


def _edge_body(hp_ref, eft_ref, wef_ref, aux1_ref, we2_ref, aux2_ref,
               out_ref, pk_ref):
    blk = hp_ref[...]
    d2 = jnp.sum(blk[:, 48:64], axis=1, keepdims=True)
    dist = jnp.sqrt(d2 + 1e-12)
    aux1 = aux1_ref[...]
    efp = lax.dot_general(eft_ref[...], wef_ref[...],
                          (((0,), (0,)), ((), ())),
                          preferred_element_type=F32)
    h = (blk[:, 0:48] + efp
         + d2 * aux1[0:1] + dist * aux1[1:2] + aux1[2:3])
    h = _gelu(_ln(h, aux1[3:4], aux1[4:5]))
    aux2 = aux2_ref[...]
    h2 = jnp.dot(h, we2_ref[...], preferred_element_type=F32) + aux2[0:1]
    e2 = _gelu(_ln(h2, aux2[1:2], aux2[2:3]))
    out_ref[...] = e2
    g = e2.shape[0] // 8
    pk_ref[...] = jnp.concatenate([e2[kk * g:(kk + 1) * g] for kk in range(8)],
                                  axis=1)


def _edge_mlp(hpart, eft, wef, aux1, we2, aux2, be=6400):
    e = hpart.shape[0]
    eh = wef.shape[1]
    de = eft.shape[0]
    oe = we2.shape[1]
    rows = be * oe // 128
    return pl.pallas_call(
        _edge_body,
        grid=(e // be,),
        in_specs=[pl.BlockSpec((be, 128), lambda i: (i, 0)),
                  pl.BlockSpec((de, be), lambda i: (0, i)),
                  pl.BlockSpec((de, eh), lambda i: (0, 0)),
                  pl.BlockSpec((5, eh), lambda i: (0, 0)),
                  pl.BlockSpec((eh, oe), lambda i: (0, 0)),
                  pl.BlockSpec((3, oe), lambda i: (0, 0))],
        out_specs=[pl.BlockSpec((be, oe), lambda i: (i, 0)),
                   pl.BlockSpec((rows, 128), lambda i: (i, 0))],
        out_shape=[jax.ShapeDtypeStruct((e, oe), F32),
                   jax.ShapeDtypeStruct((e * oe // 128, 128), F32)],
    )(hpart, eft, wef, aux1, we2, aux2)


def _sc_scatter(e_flat, col, zeros_init, oe=16, cb=80):
    n = zeros_init.shape[0]
    e = e_flat.shape[0] * e_flat.shape[1] // oe
    per_w = e // NW
    n_chunks = per_w // cb
    rpc = cb * oe // 128
    rows_per_tile = n // NS
    mesh = plsc.VectorSubcoreMesh(core_axis_name="c", subcore_axis_name="s")

    @functools.partial(
        pl.kernel, mesh=mesh,
        compiler_params=pltpu.CompilerParams(use_tc_tiling_on_sc=False),
        out_type=jax.ShapeDtypeStruct((NC * n, oe), F32),
        scratch_types=[
            pltpu.VMEM_SHARED((n, oe), F32),
            pltpu.VMEM((per_w,), jnp.int32),
            pltpu.VMEM((rpc, 128), F32),
            pltpu.VMEM((rpc, 128), F32),
            pltpu.VMEM((cb, oe), F32),
            pltpu.SemaphoreType.DMA,
            pltpu.SemaphoreType.DMA,
        ],
    )
    def k(e_hbm, col_hbm, z_hbm, out_hbm, shared, ic_all, eb0, eb1, eb16,
          sl0, sl1):
        cid = lax.axis_index("c")
        sid = lax.axis_index("s")
        wid = sid * NC + cid
        eb = (eb0, eb1)
        sl = (sl0, sl1)
        pltpu.sync_copy(col_hbm.at[pl.ds(wid * per_w, per_w)], ic_all)
        pltpu.sync_copy(z_hbm.at[pl.ds(sid * rows_per_tile, rows_per_tile)],
                        shared.at[pl.ds(sid * rows_per_tile, rows_per_tile)])
        plsc.subcore_barrier()

        def start_load(ci, s):
            base = (wid * per_w + ci * cb) * oe // 128
            pltpu.async_copy(e_hbm.at[pl.ds(base, rpc)], eb[s], sl[s])

        def wait_load(s):
            pltpu.make_async_copy(e_hbm.at[pl.ds(0, rpc)], eb[s], sl[s]).wait()

        def consume(ci, s):
            wait_load(s)
            ev = eb[s]
            npack = 128 // oe

            def row(r, _):
                for kk in range(npack):
                    eb16[r * npack + kk, :] = ev[r, pl.ds(kk * oe, oe)]
                return 0

            lax.fori_loop(0, rpc, row, 0)
            pltpu.sync_copy(eb16, shared.at[ic_all.at[pl.ds(ci * cb, cb)]],
                            add=True)

        start_load(0, 0)

        def pair(g, _):
            for b in range(2):
                ci = 2 * g + b
                s = b

                @pl.when(ci + 1 < n_chunks)
                def _():
                    start_load(ci + 1, 1 - b)

                consume(ci, s)
            return 0

        lax.fori_loop(0, n_chunks // 2, pair, 0)
        if n_chunks % 2 == 1:
            consume(n_chunks - 1, (n_chunks - 1) % 2)
        plsc.subcore_barrier()
        pltpu.sync_copy(
            shared.at[pl.ds(sid * rows_per_tile, rows_per_tile)],
            out_hbm.at[pl.ds(cid * n + sid * rows_per_tile, rows_per_tile)])

    return k(e_flat, col, zeros_init)


def _node_body(x_ref, p0_ref, p1_ref, wx_ref, wg_ref, aux1_ref, w2_ref,
               aux2_ref, out_ref):
    agg = p0_ref[...] + p1_ref[...]
    aux1 = aux1_ref[...]
    h = (jnp.dot(x_ref[...], wx_ref[...], preferred_element_type=F32,
                 precision=_PREC)
         + jnp.dot(agg, wg_ref[...], preferred_element_type=F32,
                   precision=_PREC)
         + aux1[0:1])
    h = _gelu(_ln(h, aux1[1:2], aux1[2:3]))
    aux2 = aux2_ref[...]
    h2 = jnp.dot(h, w2_ref[...], preferred_element_type=F32,
                 precision=_PREC) + aux2[0:1]
    out_ref[...] = _gelu(_ln(h2, aux2[1:2], aux2[2:3]))


def _node_mlp(x, p0, p1, wx, wg, aux1, w2, aux2, bn=2000):
    n, d = x.shape
    oe = p0.shape[1]
    nh = wx.shape[1]
    on = w2.shape[1]
    return pl.pallas_call(
        _node_body,
        grid=(n // bn,),
        in_specs=[pl.BlockSpec((bn, d), lambda i: (i, 0)),
                  pl.BlockSpec((bn, oe), lambda i: (i, 0)),
                  pl.BlockSpec((bn, oe), lambda i: (i, 0)),
                  pl.BlockSpec((d, nh), lambda i: (0, 0)),
                  pl.BlockSpec((oe, nh), lambda i: (0, 0)),
                  pl.BlockSpec((3, nh), lambda i: (0, 0)),
                  pl.BlockSpec((nh, on), lambda i: (0, 0)),
                  pl.BlockSpec((3, on), lambda i: (0, 0))],
        out_specs=pl.BlockSpec((bn, on), lambda i: (i, 0)),
        out_shape=jax.ShapeDtypeStruct((n, on), F32),
    )(x, p0, p1, wx, wg, aux1, w2, aux2)


def kernel(node_features, edge_index, edge_features,
           W_e1, b_e1, g_e1, be_e1, W_e2, b_e2, g_e2, be_e2,
           W_n1, b_n1, g_n1, be_n1, W_n2, b_n2, g_n2, be_n2):
    n, d = node_features.shape
    e = edge_index.shape[1]
    oe = W_e2.shape[1]

    ws, wt, wd = W_e1[:d], W_e1[d:2 * d], W_e1[2 * d:3 * d]
    w_d2, w_dist = W_e1[3 * d], W_e1[3 * d + 1]
    wef = W_e1[3 * d + 2:]
    wa = ws + wd
    wb = wt - wd
    row = edge_index[0]
    col = edge_index[1]

    aproj, bproj, xneg = _node_proj(node_features, wa, wb, bn=2000)
    hpart = _sc_gather(node_features, xneg, aproj, bproj, row, col)

    aux_e1 = jnp.stack([w_d2, w_dist, b_e1, g_e1, be_e1])
    aux_e2 = jnp.stack([b_e2, g_e2, be_e2])
    be = 6400
    e_out, e_pack = _edge_mlp(hpart, edge_features.T, wef, aux_e1, W_e2,
                              aux_e2, be=be)
    col_perm = col.reshape(e // be, 8, be // 8).transpose(0, 2, 1).reshape(e)

    parts = _sc_scatter(e_pack, col_perm, jnp.zeros((n, oe), F32), oe=oe)
    p0, p1 = parts[:n], parts[n:]

    aux_n1 = jnp.stack([b_n1, g_n1, be_n1])
    aux_n2 = jnp.stack([b_n2, g_n2, be_n2])
    n_out = _node_mlp(node_features, p0, p1, W_n1[:d], W_n1[d:],
                      aux_n1, W_n2, aux_n2)
    return (n_out, e_out)


# speedup vs baseline: 4.0154x; 1.1545x over previous
import functools

import jax
import jax.numpy as jnp
from jax import lax
from jax.experimental import pallas as pl
from jax.experimental.pallas import tpu as pltpu
from jax.experimental.pallas import tpu_sc as plsc

F32 = jnp.float32
_PREC = lax.Precision.HIGHEST

NC = 2
NS = 16
NW = NC * NS


def _ln(h, g, b, eps=1e-5):
    m = jnp.mean(h, axis=-1, keepdims=True)
    v = jnp.mean((h - m) ** 2, axis=-1, keepdims=True)
    return (h - m) / jnp.sqrt(v + eps) * g + b


def _gelu(h):
    return 0.5 * h * (1.0 + lax.erf(h * (2.0 ** -0.5)))


def _proj_body(x_ref, wa_ref, wb_ref, a_ref, b_ref, xn_ref):
    x = x_ref[...]
    a_ref[...] = jnp.dot(x, wa_ref[...], preferred_element_type=F32,
                         precision=_PREC)
    b_ref[...] = jnp.dot(x, wb_ref[...], preferred_element_type=F32,
                         precision=_PREC)
    xn_ref[...] = -x


def _node_proj(x, wa, wb, bn):
    n, d = x.shape
    h = wa.shape[1]
    return pl.pallas_call(
        _proj_body,
        grid=(n // bn,),
        in_specs=[pl.BlockSpec((bn, d), lambda i: (i, 0)),
                  pl.BlockSpec((d, h), lambda i: (0, 0)),
                  pl.BlockSpec((d, h), lambda i: (0, 0))],
        out_specs=[pl.BlockSpec((bn, h), lambda i: (i, 0)),
                   pl.BlockSpec((bn, h), lambda i: (i, 0)),
                   pl.BlockSpec((bn, d), lambda i: (i, 0))],
        out_shape=[jax.ShapeDtypeStruct((n, h), F32),
                   jax.ShapeDtypeStruct((n, h), F32),
                   jax.ShapeDtypeStruct((n, d), F32)],
    )(x, wa, wb)


def _sc_gather(x, xneg, aproj, bproj, row, col, cb=80):
    n, d = x.shape
    e = row.shape[0]
    eh = aproj.shape[1]
    per_w = e // NW
    n_chunks = per_w // cb
    mesh = plsc.VectorSubcoreMesh(core_axis_name="c", subcore_axis_name="s")

    @functools.partial(
        pl.kernel, mesh=mesh,
        compiler_params=pltpu.CompilerParams(use_tc_tiling_on_sc=False),
        out_type=jax.ShapeDtypeStruct((e, 128), F32),
        scratch_types=[
            pltpu.VMEM((per_w,), jnp.int32),
            pltpu.VMEM((per_w,), jnp.int32),
            pltpu.VMEM((cb, d), F32),
            pltpu.VMEM((cb, d), F32),
            pltpu.VMEM((cb, eh), F32),
            pltpu.VMEM((cb, eh), F32),
            pltpu.VMEM((cb, 128), F32),
            pltpu.VMEM((cb, 128), F32),
        ] + [pltpu.SemaphoreType.DMA] * 8,
    )
    def k(x_hbm, xn_hbm, ap_hbm, bp_hbm, row_hbm, col_hbm, hp_hbm,
          ir_all, ic_all, diff0, diff1, oh0, oh1, od0, od1,
          sa0, sa1, sb0, sb1, sc0, sc1, sd0, sd1):
        wid = lax.axis_index("s") * NC + lax.axis_index("c")
        pltpu.sync_copy(row_hbm.at[pl.ds(wid * per_w, per_w)], ir_all)
        pltpu.sync_copy(col_hbm.at[pl.ds(wid * per_w, per_w)], ic_all)

        diff = (diff0, diff1)
        oh = (oh0, oh1)
        od = (od0, od1)
        sa = (sa0, sa1)
        sb = (sb0, sb1)
        sc = (sc0, sc1)
        sd = (sd0, sd1)

        def start_ac(ci, s):
            ir = ir_all.at[pl.ds(ci * cb, cb)]
            ic = ic_all.at[pl.ds(ci * cb, cb)]
            pltpu.async_copy(x_hbm.at[ir], diff[s], sa[s])
            pltpu.async_copy(bp_hbm.at[ic], oh[s], sc[s])

        def wait_ac(s):
            ir0 = ir_all.at[pl.ds(0, cb)]
            pltpu.make_async_copy(x_hbm.at[ir0], diff[s], sa[s]).wait()
            pltpu.make_async_copy(bp_hbm.at[ir0], oh[s], sc[s]).wait()

        def start_bd(ci, s):
            ir = ir_all.at[pl.ds(ci * cb, cb)]
            ic = ic_all.at[pl.ds(ci * cb, cb)]
            pltpu.async_copy(xn_hbm.at[ic], diff[s], sb[s], add=True)
            pltpu.async_copy(ap_hbm.at[ir], oh[s], sd[s], add=True)

        def wait_bd(s):
            ir0 = ir_all.at[pl.ds(0, cb)]
            pltpu.make_async_copy(xn_hbm.at[ir0], diff[s], sb[s]).wait()
            pltpu.make_async_copy(ap_hbm.at[ir0], oh[s], sd[s]).wait()

        def consume(ci, s):
            wait_bd(s)
            dv = diff[s]
            hv = oh[s]
            ov = od[s]

            def edge(ei, _):
                acc0 = jnp.zeros((16,), F32)
                acc1 = jnp.zeros((16,), F32)
                for j in range(d // 32):
                    da = dv[ei, pl.ds(j * 32, 16)]
                    db = dv[ei, pl.ds(j * 32 + 16, 16)]
                    acc0 = acc0 + da * da
                    acc1 = acc1 + db * db
                ov[ei, pl.ds(48, 16)] = acc0 + acc1
                for j in range(eh // 16):
                    ov[ei, pl.ds(j * 16, 16)] = hv[ei, pl.ds(j * 16, 16)]
                return 0

            lax.fori_loop(0, cb, edge, 0)
            base = wid * per_w + ci * cb
            pltpu.sync_copy(ov, hp_hbm.at[pl.ds(base, cb)])

        start_ac(0, 0)
        wait_ac(0)
        start_bd(0, 0)
        start_ac(1, 1)

        def pair(g, _):
            for b in range(2):
                ci = 2 * g + b
                s = b
                so = 1 - b
                consume(ci, s)

                @pl.when(ci + 2 < n_chunks)
                def _():
                    start_ac(ci + 2, s)

                @pl.when(ci + 1 < n_chunks)
                def _():
                    wait_ac(so)
                    start_bd(ci + 1, so)
            return 0

        lax.fori_loop(0, n_chunks // 2, pair, 0)
        if n_chunks % 2 == 1:
            consume(n_chunks - 1, (n_chunks - 1) % 2)

    return k(x, xneg, aproj, bproj, row, col)


def _edge_body(hp_ref, eft_ref, wef_ref, aux1_ref, we2_ref, aux2_ref,
               out_ref, pk_ref):
    blk = hp_ref[...]
    d2 = jnp.sum(blk[:, 48:64], axis=1, keepdims=True)
    dist = jnp.sqrt(d2 + 1e-12)
    aux1 = aux1_ref[...]
    efp = lax.dot_general(eft_ref[...], wef_ref[...],
                          (((0,), (0,)), ((), ())),
                          preferred_element_type=F32)
    h = (blk[:, 0:48] + efp
         + d2 * aux1[0:1] + dist * aux1[1:2] + aux1[2:3])
    h = _gelu(_ln(h, aux1[3:4], aux1[4:5]))
    aux2 = aux2_ref[...]
    h2 = jnp.dot(h, we2_ref[...], preferred_element_type=F32) + aux2[0:1]
    e2 = _gelu(_ln(h2, aux2[1:2], aux2[2:3]))
    out_ref[...] = e2
    g = e2.shape[0] // 8
    pk_ref[...] = jnp.concatenate([e2[kk * g:(kk + 1) * g] for kk in range(8)],
                                  axis=1)


def _edge_mlp(hpart, eft, wef, aux1, we2, aux2, be=6400):
    e = hpart.shape[0]
    eh = wef.shape[1]
    de = eft.shape[0]
    oe = we2.shape[1]
    rows = be * oe // 128
    return pl.pallas_call(
        _edge_body,
        grid=(e // be,),
        in_specs=[pl.BlockSpec((be, 128), lambda i: (i, 0)),
                  pl.BlockSpec((de, be), lambda i: (0, i)),
                  pl.BlockSpec((de, eh), lambda i: (0, 0)),
                  pl.BlockSpec((5, eh), lambda i: (0, 0)),
                  pl.BlockSpec((eh, oe), lambda i: (0, 0)),
                  pl.BlockSpec((3, oe), lambda i: (0, 0))],
        out_specs=[pl.BlockSpec((be, oe), lambda i: (i, 0)),
                   pl.BlockSpec((rows, 128), lambda i: (i, 0))],
        out_shape=[jax.ShapeDtypeStruct((e, oe), F32),
                   jax.ShapeDtypeStruct((e * oe // 128, 128), F32)],
    )(hpart, eft, wef, aux1, we2, aux2)


def _sc_scatter(e_flat, col, init_tab, oe=16, cb=80):
    n = init_tab.shape[0] // NC
    e = e_flat.shape[0] * e_flat.shape[1] // oe
    per_w = e // NW
    n_chunks = per_w // cb
    rpc = cb * oe // 128
    rows_per_tile = n // NS
    mesh = plsc.VectorSubcoreMesh(core_axis_name="c", subcore_axis_name="s")

    @functools.partial(
        pl.kernel, mesh=mesh,
        compiler_params=pltpu.CompilerParams(use_tc_tiling_on_sc=False),
        out_type=jax.ShapeDtypeStruct((NC * n, oe), F32),
        scratch_types=[
            pltpu.VMEM_SHARED((n, oe), F32),
            pltpu.VMEM((per_w,), jnp.int32),
            pltpu.VMEM((rpc, 128), F32),
            pltpu.VMEM((rpc, 128), F32),
            pltpu.VMEM((cb, oe), F32),
            pltpu.SemaphoreType.DMA,
            pltpu.SemaphoreType.DMA,
        ],
    )
    def k(e_hbm, col_hbm, z_hbm, out_hbm, shared, ic_all, eb0, eb1, eb16,
          sl0, sl1):
        cid = lax.axis_index("c")
        sid = lax.axis_index("s")
        wid = sid * NC + cid
        eb = (eb0, eb1)
        sl = (sl0, sl1)
        pltpu.sync_copy(col_hbm.at[pl.ds(wid * per_w, per_w)], ic_all)
        pltpu.sync_copy(
            z_hbm.at[pl.ds(cid * n + sid * rows_per_tile, rows_per_tile)],
            shared.at[pl.ds(sid * rows_per_tile, rows_per_tile)])
        plsc.subcore_barrier()

        def start_load(ci, s):
            base = (wid * per_w + ci * cb) * oe // 128
            pltpu.async_copy(e_hbm.at[pl.ds(base, rpc)], eb[s], sl[s])

        def wait_load(s):
            pltpu.make_async_copy(e_hbm.at[pl.ds(0, rpc)], eb[s], sl[s]).wait()

        def consume(ci, s):
            wait_load(s)
            ev = eb[s]
            npack = 128 // oe

            def row(r, _):
                for kk in range(npack):
                    eb16[r * npack + kk, :] = ev[r, pl.ds(kk * oe, oe)]
                return 0

            lax.fori_loop(0, rpc, row, 0)
            pltpu.sync_copy(eb16, shared.at[ic_all.at[pl.ds(ci * cb, cb)]],
                            add=True)

        start_load(0, 0)

        def pair(g, _):
            for b in range(2):
                ci = 2 * g + b
                s = b

                @pl.when(ci + 1 < n_chunks)
                def _():
                    start_load(ci + 1, 1 - b)

                consume(ci, s)
            return 0

        lax.fori_loop(0, n_chunks // 2, pair, 0)
        if n_chunks % 2 == 1:
            consume(n_chunks - 1, (n_chunks - 1) % 2)
        plsc.subcore_barrier()
        pltpu.sync_copy(
            shared.at[pl.ds(sid * rows_per_tile, rows_per_tile)],
            out_hbm.at[pl.ds(cid * n + sid * rows_per_tile, rows_per_tile)])

    return k(e_flat, col, init_tab)


def _node_body(x_ref, p0_ref, p1_ref, wx_ref, wg_ref, aux1_ref, w2_ref,
               aux2_ref, out_ref):
    agg = p0_ref[...] + p1_ref[...]
    aux1 = aux1_ref[...]
    h = (jnp.dot(x_ref[...], wx_ref[...], preferred_element_type=F32,
                 precision=_PREC)
         + jnp.dot(agg, wg_ref[...], preferred_element_type=F32,
                   precision=_PREC)
         + aux1[0:1])
    h = _gelu(_ln(h, aux1[1:2], aux1[2:3]))
    aux2 = aux2_ref[...]
    h2 = jnp.dot(h, w2_ref[...], preferred_element_type=F32,
                 precision=_PREC) + aux2[0:1]
    out_ref[...] = _gelu(_ln(h2, aux2[1:2], aux2[2:3]))


def _node_mlp(x, p0, p1, wx, wg, aux1, w2, aux2, bn=2000):
    n, d = x.shape
    oe = p0.shape[1]
    nh = wx.shape[1]
    on = w2.shape[1]
    return pl.pallas_call(
        _node_body,
        grid=(n // bn,),
        in_specs=[pl.BlockSpec((bn, d), lambda i: (i, 0)),
                  pl.BlockSpec((bn, oe), lambda i: (i, 0)),
                  pl.BlockSpec((bn, oe), lambda i: (i, 0)),
                  pl.BlockSpec((d, nh), lambda i: (0, 0)),
                  pl.BlockSpec((oe, nh), lambda i: (0, 0)),
                  pl.BlockSpec((3, nh), lambda i: (0, 0)),
                  pl.BlockSpec((nh, on), lambda i: (0, 0)),
                  pl.BlockSpec((3, on), lambda i: (0, 0))],
        out_specs=pl.BlockSpec((bn, on), lambda i: (i, 0)),
        out_shape=jax.ShapeDtypeStruct((n, on), F32),
    )(x, p0, p1, wx, wg, aux1, w2, aux2)


def kernel(node_features, edge_index, edge_features,
           W_e1, b_e1, g_e1, be_e1, W_e2, b_e2, g_e2, be_e2,
           W_n1, b_n1, g_n1, be_n1, W_n2, b_n2, g_n2, be_n2):
    n, d = node_features.shape
    e = edge_index.shape[1]
    oe = W_e2.shape[1]

    ws, wt, wd = W_e1[:d], W_e1[d:2 * d], W_e1[2 * d:3 * d]
    w_d2, w_dist = W_e1[3 * d], W_e1[3 * d + 1]
    wef = W_e1[3 * d + 2:]
    wa = ws + wd
    wb = wt - wd
    row = edge_index[0]
    col = edge_index[1]

    aproj, bproj, xneg = _node_proj(node_features, wa, wb, bn=2000)

    aux_e1 = jnp.stack([w_d2, w_dist, b_e1, g_e1, be_e1])
    aux_e2 = jnp.stack([b_e2, g_e2, be_e2])
    be = 6400
    eft = edge_features.T
    e1 = (e * 2 // 5 // 6400) * 6400
    bounds = [(0, e1), (e1, e)]
    e_outs = []
    parts = jnp.zeros((NC * n, oe), F32)
    hp_list = []
    for lo, hi in bounds:
        hp_list.append(_sc_gather(node_features, xneg, aproj, bproj,
                                  row[lo:hi], col[lo:hi]))
    for (lo, hi), hp in zip(bounds, hp_list):
        eh_ = hi - lo
        e_out_i, e_pack_i = _edge_mlp(hp, eft[:, lo:hi], wef, aux_e1, W_e2,
                                      aux_e2, be=be)
        cp = col[lo:hi].reshape(eh_ // be, 8, be // 8)
        cp = cp.transpose(0, 2, 1).reshape(eh_)
        e_outs.append(e_out_i)
        parts = _sc_scatter(e_pack_i, cp, parts, oe=oe)
    e_out = jnp.concatenate(e_outs, axis=0)
    p0, p1 = parts[:n], parts[n:]

    aux_n1 = jnp.stack([b_n1, g_n1, be_n1])
    aux_n2 = jnp.stack([b_n2, g_n2, be_n2])
    n_out = _node_mlp(node_features, p0, p1, W_n1[:d], W_n1[d:],
                      aux_n1, W_n2, aux_n2)
    return (n_out, e_out)


# speedup vs baseline: 4.3184x; 1.0755x over previous
import functools

import jax
import jax.numpy as jnp
from jax import lax
from jax.experimental import pallas as pl
from jax.experimental.pallas import tpu as pltpu
from jax.experimental.pallas import tpu_sc as plsc

F32 = jnp.float32
_PREC = lax.Precision.HIGHEST

NC = 2
NS = 16
NW = NC * NS


def _ln(h, g, b, eps=1e-5):
    m = jnp.mean(h, axis=-1, keepdims=True)
    v = jnp.mean((h - m) ** 2, axis=-1, keepdims=True)
    return (h - m) / jnp.sqrt(v + eps) * g + b


def _gelu(h):
    return 0.5 * h * (1.0 + lax.erf(h * (2.0 ** -0.5)))


def _proj_body(x_ref, wa_ref, wb_ref, a_ref, b_ref, xn_ref):
    x = x_ref[...]
    a_ref[...] = jnp.dot(x, wa_ref[...], preferred_element_type=F32,
                         precision=_PREC)
    b_ref[...] = jnp.dot(x, wb_ref[...], preferred_element_type=F32,
                         precision=_PREC)
    xn_ref[...] = -x


def _node_proj(x, wa, wb, bn):
    n, d = x.shape
    h = wa.shape[1]
    return pl.pallas_call(
        _proj_body,
        grid=(n // bn,),
        in_specs=[pl.BlockSpec((bn, d), lambda i: (i, 0)),
                  pl.BlockSpec((d, h), lambda i: (0, 0)),
                  pl.BlockSpec((d, h), lambda i: (0, 0))],
        out_specs=[pl.BlockSpec((bn, h), lambda i: (i, 0)),
                   pl.BlockSpec((bn, h), lambda i: (i, 0)),
                   pl.BlockSpec((bn, d), lambda i: (i, 0))],
        out_shape=[jax.ShapeDtypeStruct((n, h), F32),
                   jax.ShapeDtypeStruct((n, h), F32),
                   jax.ShapeDtypeStruct((n, d), F32)],
    )(x, wa, wb)


def _sc_gather(x, xneg, aproj, bproj, row, col, cb=80):
    n, d = x.shape
    e = row.shape[0]
    eh = aproj.shape[1]
    per_w = e // NW
    n_chunks = per_w // cb
    mesh = plsc.VectorSubcoreMesh(core_axis_name="c", subcore_axis_name="s")

    @functools.partial(
        pl.kernel, mesh=mesh,
        compiler_params=pltpu.CompilerParams(use_tc_tiling_on_sc=False),
        out_type=jax.ShapeDtypeStruct((e, 128), F32),
        scratch_types=[
            pltpu.VMEM((per_w,), jnp.int32),
            pltpu.VMEM((per_w,), jnp.int32),
            pltpu.VMEM((cb, d), F32),
            pltpu.VMEM((cb, d), F32),
            pltpu.VMEM((cb, eh), F32),
            pltpu.VMEM((cb, eh), F32),
            pltpu.VMEM((cb, 128), F32),
            pltpu.VMEM((cb, 128), F32),
        ] + [pltpu.SemaphoreType.DMA] * 8,
    )
    def k(x_hbm, xn_hbm, ap_hbm, bp_hbm, row_hbm, col_hbm, hp_hbm,
          ir_all, ic_all, diff0, diff1, oh0, oh1, od0, od1,
          sa0, sa1, sb0, sb1, sc0, sc1, sd0, sd1):
        wid = lax.axis_index("s") * NC + lax.axis_index("c")
        pltpu.sync_copy(row_hbm.at[pl.ds(wid * per_w, per_w)], ir_all)
        pltpu.sync_copy(col_hbm.at[pl.ds(wid * per_w, per_w)], ic_all)

        diff = (diff0, diff1)
        oh = (oh0, oh1)
        od = (od0, od1)
        sa = (sa0, sa1)
        sb = (sb0, sb1)
        sc = (sc0, sc1)
        sd = (sd0, sd1)

        def start_ac(ci, s):
            ir = ir_all.at[pl.ds(ci * cb, cb)]
            ic = ic_all.at[pl.ds(ci * cb, cb)]
            pltpu.async_copy(x_hbm.at[ir], diff[s], sa[s])
            pltpu.async_copy(bp_hbm.at[ic], oh[s], sc[s])

        def wait_ac(s):
            ir0 = ir_all.at[pl.ds(0, cb)]
            pltpu.make_async_copy(x_hbm.at[ir0], diff[s], sa[s]).wait()
            pltpu.make_async_copy(bp_hbm.at[ir0], oh[s], sc[s]).wait()

        def start_bd(ci, s):
            ir = ir_all.at[pl.ds(ci * cb, cb)]
            ic = ic_all.at[pl.ds(ci * cb, cb)]
            pltpu.async_copy(xn_hbm.at[ic], diff[s], sb[s], add=True)
            pltpu.async_copy(ap_hbm.at[ir], oh[s], sd[s], add=True)

        def wait_bd(s):
            ir0 = ir_all.at[pl.ds(0, cb)]
            pltpu.make_async_copy(xn_hbm.at[ir0], diff[s], sb[s]).wait()
            pltpu.make_async_copy(ap_hbm.at[ir0], oh[s], sd[s]).wait()

        def consume(ci, s):
            wait_bd(s)
            dv = diff[s]
            hv = oh[s]
            ov = od[s]

            def edge(ei, _):
                acc0 = jnp.zeros((16,), F32)
                acc1 = jnp.zeros((16,), F32)
                for j in range(d // 32):
                    da = dv[ei, pl.ds(j * 32, 16)]
                    db = dv[ei, pl.ds(j * 32 + 16, 16)]
                    acc0 = acc0 + da * da
                    acc1 = acc1 + db * db
                ov[ei, pl.ds(48, 16)] = acc0 + acc1
                for j in range(eh // 16):
                    ov[ei, pl.ds(j * 16, 16)] = hv[ei, pl.ds(j * 16, 16)]
                return 0

            lax.fori_loop(0, cb, edge, 0)
            base = wid * per_w + ci * cb
            pltpu.sync_copy(ov, hp_hbm.at[pl.ds(base, cb)])

        start_ac(0, 0)
        wait_ac(0)
        start_bd(0, 0)
        start_ac(1, 1)

        def pair(g, _):
            for b in range(2):
                ci = 2 * g + b
                s = b
                so = 1 - b
                consume(ci, s)

                @pl.when(ci + 2 < n_chunks)
                def _():
                    start_ac(ci + 2, s)

                @pl.when(ci + 1 < n_chunks)
                def _():
                    wait_ac(so)
                    start_bd(ci + 1, so)
            return 0

        lax.fori_loop(0, n_chunks // 2, pair, 0)
        if n_chunks % 2 == 1:
            consume(n_chunks - 1, (n_chunks - 1) % 2)

    return k(x, xneg, aproj, bproj, row, col)


def _edge_body(hp_ref, eft_ref, wef_ref, aux1_ref, we2_ref, aux2_ref,
               out_ref, pk_ref):
    blk = hp_ref[...]
    d2 = jnp.sum(blk[:, 48:64], axis=1, keepdims=True)
    dist = jnp.sqrt(d2 + 1e-12)
    aux1 = aux1_ref[...]
    efp = lax.dot_general(eft_ref[...], wef_ref[...],
                          (((0,), (0,)), ((), ())),
                          preferred_element_type=F32)
    h = (blk[:, 0:48] + efp
         + d2 * aux1[0:1] + dist * aux1[1:2] + aux1[2:3])
    h = _gelu(_ln(h, aux1[3:4], aux1[4:5]))
    aux2 = aux2_ref[...]
    h2 = jnp.dot(h, we2_ref[...], preferred_element_type=F32) + aux2[0:1]
    e2 = _gelu(_ln(h2, aux2[1:2], aux2[2:3]))
    out_ref[...] = e2
    g = e2.shape[0] // 8
    pk_ref[...] = jnp.concatenate([e2[kk * g:(kk + 1) * g] for kk in range(8)],
                                  axis=1)


def _edge_mlp(hpart, eft, wef, aux1, we2, aux2, be=6400):
    e = hpart.shape[0]
    eh = wef.shape[1]
    de = eft.shape[0]
    oe = we2.shape[1]
    rows = be * oe // 128
    return pl.pallas_call(
        _edge_body,
        grid=(e // be,),
        in_specs=[pl.BlockSpec((be, 128), lambda i: (i, 0)),
                  pl.BlockSpec((de, be), lambda i: (0, i)),
                  pl.BlockSpec((de, eh), lambda i: (0, 0)),
                  pl.BlockSpec((5, eh), lambda i: (0, 0)),
                  pl.BlockSpec((eh, oe), lambda i: (0, 0)),
                  pl.BlockSpec((3, oe), lambda i: (0, 0))],
        out_specs=[pl.BlockSpec((be, oe), lambda i: (i, 0)),
                   pl.BlockSpec((rows, 128), lambda i: (i, 0))],
        out_shape=[jax.ShapeDtypeStruct((e, oe), F32),
                   jax.ShapeDtypeStruct((e * oe // 128, 128), F32)],
    )(hpart, eft, wef, aux1, we2, aux2)


def _sc_scatter(e_flat, col, init_tab, oe=16, cb=80):
    n = init_tab.shape[0] // NC
    e = e_flat.shape[0] * e_flat.shape[1] // oe
    per_w = e // NW
    n_chunks = per_w // cb
    rpc = cb * oe // 128
    rows_per_tile = n // NS
    mesh = plsc.VectorSubcoreMesh(core_axis_name="c", subcore_axis_name="s")

    @functools.partial(
        pl.kernel, mesh=mesh,
        compiler_params=pltpu.CompilerParams(use_tc_tiling_on_sc=False),
        out_type=jax.ShapeDtypeStruct((NC * n, oe), F32),
        scratch_types=[
            pltpu.VMEM_SHARED((n, oe), F32),
            pltpu.VMEM((per_w,), jnp.int32),
            pltpu.VMEM((rpc, 128), F32),
            pltpu.VMEM((rpc, 128), F32),
            pltpu.VMEM((cb, oe), F32),
            pltpu.SemaphoreType.DMA,
            pltpu.SemaphoreType.DMA,
        ],
    )
    def k(e_hbm, col_hbm, z_hbm, out_hbm, shared, ic_all, eb0, eb1, eb16,
          sl0, sl1):
        cid = lax.axis_index("c")
        sid = lax.axis_index("s")
        wid = sid * NC + cid
        eb = (eb0, eb1)
        sl = (sl0, sl1)
        pltpu.sync_copy(col_hbm.at[pl.ds(wid * per_w, per_w)], ic_all)
        pltpu.sync_copy(
            z_hbm.at[pl.ds(cid * n + sid * rows_per_tile, rows_per_tile)],
            shared.at[pl.ds(sid * rows_per_tile, rows_per_tile)])
        plsc.subcore_barrier()

        def start_load(ci, s):
            base = (wid * per_w + ci * cb) * oe // 128
            pltpu.async_copy(e_hbm.at[pl.ds(base, rpc)], eb[s], sl[s])

        def wait_load(s):
            pltpu.make_async_copy(e_hbm.at[pl.ds(0, rpc)], eb[s], sl[s]).wait()

        def consume(ci, s):
            wait_load(s)
            ev = eb[s]
            npack = 128 // oe

            def row(r, _):
                for kk in range(npack):
                    eb16[r * npack + kk, :] = ev[r, pl.ds(kk * oe, oe)]
                return 0

            lax.fori_loop(0, rpc, row, 0)
            pltpu.sync_copy(eb16, shared.at[ic_all.at[pl.ds(ci * cb, cb)]],
                            add=True)

        start_load(0, 0)

        def pair(g, _):
            for b in range(2):
                ci = 2 * g + b
                s = b

                @pl.when(ci + 1 < n_chunks)
                def _():
                    start_load(ci + 1, 1 - b)

                consume(ci, s)
            return 0

        lax.fori_loop(0, n_chunks // 2, pair, 0)
        if n_chunks % 2 == 1:
            consume(n_chunks - 1, (n_chunks - 1) % 2)
        plsc.subcore_barrier()
        pltpu.sync_copy(
            shared.at[pl.ds(sid * rows_per_tile, rows_per_tile)],
            out_hbm.at[pl.ds(cid * n + sid * rows_per_tile, rows_per_tile)])

    return k(e_flat, col, init_tab)


def _node_body(x_ref, p0_ref, p1_ref, wx_ref, wg_ref, aux1_ref, w2_ref,
               aux2_ref, out_ref):
    agg = p0_ref[...] + p1_ref[...]
    aux1 = aux1_ref[...]
    h = (jnp.dot(x_ref[...], wx_ref[...], preferred_element_type=F32,
                 precision=_PREC)
         + jnp.dot(agg, wg_ref[...], preferred_element_type=F32,
                   precision=_PREC)
         + aux1[0:1])
    h = _gelu(_ln(h, aux1[1:2], aux1[2:3]))
    aux2 = aux2_ref[...]
    h2 = jnp.dot(h, w2_ref[...], preferred_element_type=F32,
                 precision=_PREC) + aux2[0:1]
    out_ref[...] = _gelu(_ln(h2, aux2[1:2], aux2[2:3]))


def _node_mlp(x, p0, p1, wx, wg, aux1, w2, aux2, bn=2000):
    n, d = x.shape
    oe = p0.shape[1]
    nh = wx.shape[1]
    on = w2.shape[1]
    return pl.pallas_call(
        _node_body,
        grid=(n // bn,),
        in_specs=[pl.BlockSpec((bn, d), lambda i: (i, 0)),
                  pl.BlockSpec((bn, oe), lambda i: (i, 0)),
                  pl.BlockSpec((bn, oe), lambda i: (i, 0)),
                  pl.BlockSpec((d, nh), lambda i: (0, 0)),
                  pl.BlockSpec((oe, nh), lambda i: (0, 0)),
                  pl.BlockSpec((3, nh), lambda i: (0, 0)),
                  pl.BlockSpec((nh, on), lambda i: (0, 0)),
                  pl.BlockSpec((3, on), lambda i: (0, 0))],
        out_specs=pl.BlockSpec((bn, on), lambda i: (i, 0)),
        out_shape=jax.ShapeDtypeStruct((n, on), F32),
    )(x, p0, p1, wx, wg, aux1, w2, aux2)


def kernel(node_features, edge_index, edge_features,
           W_e1, b_e1, g_e1, be_e1, W_e2, b_e2, g_e2, be_e2,
           W_n1, b_n1, g_n1, be_n1, W_n2, b_n2, g_n2, be_n2):
    n, d = node_features.shape
    e = edge_index.shape[1]
    oe = W_e2.shape[1]

    ws, wt, wd = W_e1[:d], W_e1[d:2 * d], W_e1[2 * d:3 * d]
    w_d2, w_dist = W_e1[3 * d], W_e1[3 * d + 1]
    wef = W_e1[3 * d + 2:]
    wa = ws + wd
    wb = wt - wd
    row = edge_index[0]
    col = edge_index[1]

    aproj, bproj, xneg = _node_proj(node_features, wa, wb, bn=2000)

    aux_e1 = jnp.stack([w_d2, w_dist, b_e1, g_e1, be_e1])
    aux_e2 = jnp.stack([b_e2, g_e2, be_e2])
    be = 6400
    eft = edge_features.T
    cut1 = 89600 * e // 320000
    cut2 = cut1 + 115200 * e // 320000
    bounds = [(0, cut1), (cut1, cut2), (cut2, e)]
    e_outs = []
    parts = jnp.zeros((NC * n, oe), F32)
    hp_list = []
    for lo, hi in bounds:
        hp_list.append(_sc_gather(node_features, xneg, aproj, bproj,
                                  row[lo:hi], col[lo:hi]))
    for (lo, hi), hp in zip(bounds, hp_list):
        eh_ = hi - lo
        e_out_i, e_pack_i = _edge_mlp(hp, eft[:, lo:hi], wef, aux_e1, W_e2,
                                      aux_e2, be=be)
        cp = col[lo:hi].reshape(eh_ // be, 8, be // 8)
        cp = cp.transpose(0, 2, 1).reshape(eh_)
        e_outs.append(e_out_i)
        parts = _sc_scatter(e_pack_i, cp, parts, oe=oe)
    e_out = jnp.concatenate(e_outs, axis=0)
    p0, p1 = parts[:n], parts[n:]

    aux_n1 = jnp.stack([b_n1, g_n1, be_n1])
    aux_n2 = jnp.stack([b_n2, g_n2, be_n2])
    n_out = _node_mlp(node_features, p0, p1, W_n1[:d], W_n1[d:],
                      aux_n1, W_n2, aux_n2)
    return (n_out, e_out)
